# baseline (speedup 1.0000x reference)
import functools

import jax
import jax.numpy as jnp
from jax import lax
from jax.experimental import pallas as pl
from jax.experimental.pallas import tpu as pltpu
from jax.experimental.pallas import tpu_sc as plsc

N = 10000
E = 320000
NF = 128
ND = 20
HARD_CUTOFF = 6.5

NW = 32
CHUNK = 128
NCHUNKS = E // CHUNK
STEPS = (NCHUNKS + NW - 1) // NW

BB = 512
NB = E // BB

NFULL = N // CHUNK
NREM = N - NFULL * CHUNK


def _sc_gather_body(x_hbm, ps_hbm, g_hbm, idx_v, rows_v, sem):
    wid = lax.axis_index("c") * 16 + lax.axis_index("s")

    def step(j, carry):
        c = j * NW + wid

        @pl.when(c < NCHUNKS)
        def _():
            base = c * CHUNK
            pltpu.sync_copy(ps_hbm.at[pl.ds(base, CHUNK)], idx_v)
            pltpu.async_copy(x_hbm.at[idx_v], rows_v, sem).wait()
            pltpu.sync_copy(rows_v, g_hbm.at[pl.ds(base, CHUNK)])

        return carry

    lax.fori_loop(0, STEPS, step, 0)


def _sc_scatter_body(z_hbm, pf_hbm, out_hbm, idx_v, rows_v, acc_sh):
    cid = lax.axis_index("c")
    sid = lax.axis_index("s")
    wid = cid * 16 + sid

    zeros16 = jnp.zeros((16,), jnp.float32)

    def zstep(i, carry):
        r = i // (NF // 16)
        col = (i % (NF // 16)) * 16
        rows_v[r, pl.ds(col, 16)] = zeros16
        return carry

    lax.fori_loop(0, CHUNK * (NF // 16), zstep, 0)

    for i in range((NFULL + 15) // 16):
        blk = sid + i * 16

        @pl.when(blk < NFULL)
        def _():
            pltpu.sync_copy(rows_v, acc_sh.at[pl.ds(blk * CHUNK, CHUNK)])

    @pl.when(sid == 0)
    def _():
        pltpu.sync_copy(rows_v.at[pl.ds(0, NREM)],
                        acc_sh.at[pl.ds(NFULL * CHUNK, NREM)])

    plsc.subcore_barrier()

    def step(j, carry):
        c = j * NW + wid

        @pl.when(c < NCHUNKS)
        def _():
            base = c * CHUNK
            pltpu.sync_copy(pf_hbm.at[pl.ds(base, CHUNK)], idx_v)
            pltpu.sync_copy(z_hbm.at[pl.ds(base, CHUNK)], rows_v)
            pltpu.sync_copy(rows_v, acc_sh.at[idx_v], add=True)

        return carry

    lax.fori_loop(0, STEPS, step, 0)
    plsc.subcore_barrier()

    for i in range((NFULL + 15) // 16):
        blk = sid + i * 16

        @pl.when(blk < NFULL)
        def _():
            pltpu.sync_copy(acc_sh.at[pl.ds(blk * CHUNK, CHUNK)], rows_v)
            pltpu.sync_copy(rows_v, out_hbm.at[pl.ds(cid * N + blk * CHUNK, CHUNK)])

    @pl.when(sid == 0)
    def _():
        pltpu.sync_copy(acc_sh.at[pl.ds(NFULL * CHUNK, NREM)],
                        rows_v.at[pl.ds(0, NREM)])
        pltpu.sync_copy(rows_v.at[pl.ds(0, NREM)],
                        out_hbm.at[pl.ds(cid * N + NFULL * CHUNK, NREM)])


def _tc_z_body(g_ref, d_ref, w_ref, mu_ref, sg_ref, z_ref):
    g = g_ref[...].astype(jnp.bfloat16)
    d = d_ref[0]
    inv = 1.0 / d
    cut = jnp.where(
        d < HARD_CUTOFF,
        jnp.cos(d * (jnp.pi / (2.0 * HARD_CUTOFF))) ** 2,
        0.0,
    )
    h = jnp.dot(g, w_ref[...],
                preferred_element_type=jnp.float32).astype(jnp.bfloat16)
    acc = jnp.zeros((BB, NF), jnp.float32)
    for k in range(ND):
        t = (inv - mu_ref[0, k]) / sg_ref[0, k]
        s_k = jnp.exp(-0.5 * t * t) * cut
        acc = acc + s_k * h[:, k * NF:(k + 1) * NF].astype(jnp.float32)
    z_ref[...] = acc


TD = 400


def _tc_out_body(p_ref, x_ref, w_ref, b_ref, o_ref):
    s = jnp.dot(x_ref[...], w_ref[...], preferred_element_type=jnp.float32)
    o_ref[...] = p_ref[0] + p_ref[1] + s + b_ref[...]


def kernel(in_features, pair_first, pair_second, dist_pairs, mu, sigma,
           int_weights, self_W, self_b):
    ps = pair_second.astype(jnp.int32)
    pf = pair_first.astype(jnp.int32)
    x = in_features.astype(jnp.float32)

    mesh = plsc.VectorSubcoreMesh(core_axis_name="c", subcore_axis_name="s")

    gather = pl.kernel(
        _sc_gather_body,
        out_type=jax.ShapeDtypeStruct((E, NF), jnp.float32),
        mesh=mesh,
        scratch_types=[
            pltpu.VMEM((CHUNK,), jnp.int32),
            pltpu.VMEM((CHUNK, NF), jnp.float32),
            pltpu.SemaphoreType.DMA,
        ],
    )
    g = gather(x, ps)

    wm = jnp.transpose(int_weights, (2, 0, 1)).reshape(NF, ND * NF)
    wm = wm.astype(jnp.bfloat16)
    dist4 = dist_pairs.astype(jnp.float32).reshape(NB, BB, 1)
    mu2 = mu.astype(jnp.float32).reshape(1, ND)
    sg2 = sigma.astype(jnp.float32).reshape(1, ND)
    z = pl.pallas_call(
        _tc_z_body,
        grid=(NB,),
        in_specs=[
            pl.BlockSpec((BB, NF), lambda b: (b, 0)),
            pl.BlockSpec((1, BB, 1), lambda b: (b, 0, 0)),
            pl.BlockSpec((NF, ND * NF), lambda b: (0, 0)),
            pl.BlockSpec(memory_space=pltpu.SMEM),
            pl.BlockSpec(memory_space=pltpu.SMEM),
        ],
        out_specs=pl.BlockSpec((BB, NF), lambda b: (b, 0)),
        out_shape=jax.ShapeDtypeStruct((E, NF), jnp.float32),
    )(g, dist4, wm, mu2, sg2)

    scatter = pl.kernel(
        _sc_scatter_body,
        out_type=jax.ShapeDtypeStruct((2 * N, NF), jnp.float32),
        mesh=mesh,
        scratch_types=[
            pltpu.VMEM((CHUNK,), jnp.int32),
            pltpu.VMEM((CHUNK, NF), jnp.float32),
            pltpu.VMEM_SHARED((N, NF), jnp.float32),
        ],
    )
    partial = scatter(z, pf).reshape(2, N, NF)

    swt = jnp.transpose(self_W, (1, 0)).astype(jnp.float32)
    b2 = self_b.astype(jnp.float32).reshape(1, NF)
    out = pl.pallas_call(
        _tc_out_body,
        grid=(N // TD,),
        in_specs=[
            pl.BlockSpec((2, TD, NF), lambda b: (0, b, 0)),
            pl.BlockSpec((TD, NF), lambda b: (b, 0)),
            pl.BlockSpec((NF, NF), lambda b: (0, 0)),
            pl.BlockSpec((1, NF), lambda b: (0, 0)),
        ],
        out_specs=pl.BlockSpec((TD, NF), lambda b: (b, 0)),
        out_shape=jax.ShapeDtypeStruct((N, NF), jnp.float32),
    )(partial, x, swt, b2)
    return out


# speedup vs baseline: 1.4710x; 1.4710x over previous
import functools

import jax
import jax.numpy as jnp
from jax import lax
from jax.experimental import pallas as pl
from jax.experimental.pallas import tpu as pltpu
from jax.experimental.pallas import tpu_sc as plsc

N = 10000
E = 320000
NF = 128
ND = 20
HARD_CUTOFF = 6.5

NW = 32
CHUNK = 128
NCHUNKS = E // CHUNK
STEPS = (NCHUNKS + NW - 1) // NW

BB = 512
NB = E // BB

NFULL = N // CHUNK
NREM = N - NFULL * CHUNK


def _sc_gather_body(x_hbm, ps_hbm, g_hbm, idx_v, rows_v, sem):
    wid = lax.axis_index("c") * 16 + lax.axis_index("s")

    def step(j, carry):
        c = j * NW + wid

        @pl.when(c < NCHUNKS)
        def _():
            base = c * CHUNK
            pltpu.sync_copy(ps_hbm.at[pl.ds(base, CHUNK)], idx_v)
            pltpu.async_copy(x_hbm.at[idx_v], rows_v, sem).wait()
            pltpu.sync_copy(rows_v, g_hbm.at[pl.ds(base, CHUNK)])

        return carry

    lax.fori_loop(0, STEPS, step, 0)


def _sc_scatter_body(z_hbm, pf_hbm, out_hbm, idx_v, rows_v, acc_sh):
    cid = lax.axis_index("c")
    sid = lax.axis_index("s")
    wid = cid * 16 + sid

    zeros16 = jnp.zeros((16,), jnp.float32)

    def zstep(i, carry):
        r = i // (NF // 16)
        col = (i % (NF // 16)) * 16
        rows_v[r, pl.ds(col, 16)] = zeros16
        return carry

    lax.fori_loop(0, CHUNK * (NF // 16), zstep, 0)

    for i in range((NFULL + 15) // 16):
        blk = sid + i * 16

        @pl.when(blk < NFULL)
        def _():
            pltpu.sync_copy(rows_v, acc_sh.at[pl.ds(blk * CHUNK, CHUNK)])

    @pl.when(sid == 0)
    def _():
        pltpu.sync_copy(rows_v.at[pl.ds(0, NREM)],
                        acc_sh.at[pl.ds(NFULL * CHUNK, NREM)])

    plsc.subcore_barrier()

    def step(j, carry):
        c = j * NW + wid

        @pl.when(c < NCHUNKS)
        def _():
            base = c * CHUNK
            pltpu.sync_copy(pf_hbm.at[pl.ds(base, CHUNK)], idx_v)
            pltpu.sync_copy(z_hbm.at[pl.ds(base, CHUNK)], rows_v)
            pltpu.sync_copy(rows_v, acc_sh.at[idx_v], add=True)

        return carry

    lax.fori_loop(0, STEPS, step, 0)
    plsc.subcore_barrier()

    for i in range((NFULL + 15) // 16):
        blk = sid + i * 16

        @pl.when(blk < NFULL)
        def _():
            pltpu.sync_copy(acc_sh.at[pl.ds(blk * CHUNK, CHUNK)], rows_v)
            pltpu.sync_copy(rows_v, out_hbm.at[pl.ds(cid * N + blk * CHUNK, CHUNK)])

    @pl.when(sid == 0)
    def _():
        pltpu.sync_copy(acc_sh.at[pl.ds(NFULL * CHUNK, NREM)],
                        rows_v.at[pl.ds(0, NREM)])
        pltpu.sync_copy(rows_v.at[pl.ds(0, NREM)],
                        out_hbm.at[pl.ds(cid * N + NFULL * CHUNK, NREM)])


def _tc_z_body(g_ref, d_ref, w_ref, mu_ref, sg_ref, z_ref):
    g = g_ref[...]
    d = d_ref[0]
    inv = 1.0 / d
    cut = jnp.where(
        d < HARD_CUTOFF,
        jnp.cos(d * (jnp.pi / (2.0 * HARD_CUTOFF))) ** 2,
        0.0,
    )
    rows = []
    for k in range(ND):
        t = (inv - mu_ref[0, k]) / sg_ref[0, k]
        rows.append(jnp.exp(-0.5 * t * t) * cut)
    st = jnp.transpose(jnp.concatenate(rows, axis=0))
    h = jnp.dot(g, w_ref[...], preferred_element_type=jnp.float32)
    acc = jnp.zeros((BB, NF), jnp.float32)
    for k in range(ND):
        acc = acc + st[:, k:k + 1] * h[:, k * NF:(k + 1) * NF]
    z_ref[...] = acc


TD = 400


def _tc_out_body(p_ref, x_ref, w_ref, b_ref, o_ref):
    s = jnp.dot(x_ref[...], w_ref[...], preferred_element_type=jnp.float32)
    o_ref[...] = p_ref[0] + p_ref[1] + s + b_ref[...]


def kernel(in_features, pair_first, pair_second, dist_pairs, mu, sigma,
           int_weights, self_W, self_b):
    ps = pair_second.astype(jnp.int32)
    pf = pair_first.astype(jnp.int32)
    x = in_features.astype(jnp.float32)

    mesh = plsc.VectorSubcoreMesh(core_axis_name="c", subcore_axis_name="s")

    gather = pl.kernel(
        _sc_gather_body,
        out_type=jax.ShapeDtypeStruct((E, NF), jnp.float32),
        mesh=mesh,
        scratch_types=[
            pltpu.VMEM((CHUNK,), jnp.int32),
            pltpu.VMEM((CHUNK, NF), jnp.float32),
            pltpu.SemaphoreType.DMA,
        ],
    )
    g = gather(x, ps)

    wm = jnp.transpose(int_weights, (2, 0, 1)).reshape(NF, ND * NF)
    dist4 = dist_pairs.astype(jnp.float32).reshape(NB, 1, BB)
    mu2 = mu.astype(jnp.float32).reshape(1, ND)
    sg2 = sigma.astype(jnp.float32).reshape(1, ND)
    z = pl.pallas_call(
        _tc_z_body,
        grid=(NB,),
        in_specs=[
            pl.BlockSpec((BB, NF), lambda b: (b, 0)),
            pl.BlockSpec((1, 1, BB), lambda b: (b, 0, 0)),
            pl.BlockSpec((NF, ND * NF), lambda b: (0, 0)),
            pl.BlockSpec(memory_space=pltpu.SMEM),
            pl.BlockSpec(memory_space=pltpu.SMEM),
        ],
        out_specs=pl.BlockSpec((BB, NF), lambda b: (b, 0)),
        out_shape=jax.ShapeDtypeStruct((E, NF), jnp.float32),
    )(g, dist4, wm, mu2, sg2)

    scatter = pl.kernel(
        _sc_scatter_body,
        out_type=jax.ShapeDtypeStruct((2 * N, NF), jnp.float32),
        mesh=mesh,
        scratch_types=[
            pltpu.VMEM((CHUNK,), jnp.int32),
            pltpu.VMEM((CHUNK, NF), jnp.float32),
            pltpu.VMEM_SHARED((N, NF), jnp.float32),
        ],
    )
    partial = scatter(z, pf).reshape(2, N, NF)

    swt = jnp.transpose(self_W, (1, 0)).astype(jnp.float32)
    b2 = self_b.astype(jnp.float32).reshape(1, NF)
    out = pl.pallas_call(
        _tc_out_body,
        grid=(N // TD,),
        in_specs=[
            pl.BlockSpec((2, TD, NF), lambda b: (0, b, 0)),
            pl.BlockSpec((TD, NF), lambda b: (b, 0)),
            pl.BlockSpec((NF, NF), lambda b: (0, 0)),
            pl.BlockSpec((1, NF), lambda b: (0, 0)),
        ],
        out_specs=pl.BlockSpec((TD, NF), lambda b: (b, 0)),
        out_shape=jax.ShapeDtypeStruct((N, NF), jnp.float32),
    )(partial, x, swt, b2)
    return out


# speedup vs baseline: 2.0797x; 1.4138x over previous
import functools

import jax
import jax.numpy as jnp
from jax import lax
from jax.experimental import pallas as pl
from jax.experimental.pallas import tpu as pltpu
from jax.experimental.pallas import tpu_sc as plsc

N = 10000
E = 320000
NF = 128
ND = 20
HARD_CUTOFF = 6.5

NW = 32
CHUNK = 128
NCHUNKS = E // CHUNK
STEPS = (NCHUNKS + NW - 1) // NW

BB = 512
NB = E // BB

NFULL = N // CHUNK
NREM = N - NFULL * CHUNK


def _sc_gather_body(x_hbm, ps_hbm, g_hbm, idx_v, rows_v, sem):
    wid = lax.axis_index("c") * 16 + lax.axis_index("s")

    def step(j, carry):
        c = j * NW + wid

        @pl.when(c < NCHUNKS)
        def _():
            base = c * CHUNK
            pltpu.sync_copy(ps_hbm.at[pl.ds(base, CHUNK)], idx_v)
            pltpu.async_copy(x_hbm.at[idx_v], rows_v, sem).wait()
            pltpu.sync_copy(rows_v, g_hbm.at[pl.ds(base, CHUNK)])

        return carry

    lax.fori_loop(0, STEPS, step, 0)


def _sc_scatter_body(z_hbm, pf_hbm, out_hbm, idx_v, rows_v, acc_sh):
    cid = lax.axis_index("c")
    sid = lax.axis_index("s")
    wid = cid * 16 + sid

    zeros16 = jnp.zeros((16,), jnp.float32)

    def zstep(i, carry):
        r = i // (NF // 16)
        col = (i % (NF // 16)) * 16
        rows_v[r, pl.ds(col, 16)] = zeros16
        return carry

    lax.fori_loop(0, CHUNK * (NF // 16), zstep, 0)

    for i in range((NFULL + 15) // 16):
        blk = sid + i * 16

        @pl.when(blk < NFULL)
        def _():
            pltpu.sync_copy(rows_v, acc_sh.at[pl.ds(blk * CHUNK, CHUNK)])

    @pl.when(sid == 0)
    def _():
        pltpu.sync_copy(rows_v.at[pl.ds(0, NREM)],
                        acc_sh.at[pl.ds(NFULL * CHUNK, NREM)])

    plsc.subcore_barrier()

    def step(j, carry):
        c = j * NW + wid

        @pl.when(c < NCHUNKS)
        def _():
            base = c * CHUNK
            pltpu.sync_copy(pf_hbm.at[pl.ds(base, CHUNK)], idx_v)
            pltpu.sync_copy(z_hbm.at[pl.ds(base, CHUNK)], rows_v)
            pltpu.sync_copy(rows_v, acc_sh.at[idx_v], add=True)

        return carry

    lax.fori_loop(0, STEPS, step, 0)
    plsc.subcore_barrier()

    for i in range((NFULL + 15) // 16):
        blk = sid + i * 16

        @pl.when(blk < NFULL)
        def _():
            pltpu.sync_copy(acc_sh.at[pl.ds(blk * CHUNK, CHUNK)], rows_v)
            pltpu.sync_copy(rows_v, out_hbm.at[pl.ds(cid * N + blk * CHUNK, CHUNK)])

    @pl.when(sid == 0)
    def _():
        pltpu.sync_copy(acc_sh.at[pl.ds(NFULL * CHUNK, NREM)],
                        rows_v.at[pl.ds(0, NREM)])
        pltpu.sync_copy(rows_v.at[pl.ds(0, NREM)],
                        out_hbm.at[pl.ds(cid * N + NFULL * CHUNK, NREM)])


def _tc_z_body(g_ref, d_ref, w_ref, mu_ref, sg_ref, z_ref):
    gt = jnp.transpose(g_ref[...])
    d = d_ref[0]
    inv = 1.0 / d
    cut = jnp.where(
        d < HARD_CUTOFF,
        jnp.cos(d * (jnp.pi / (2.0 * HARD_CUTOFF))) ** 2,
        0.0,
    )
    rows = []
    for k in range(ND):
        t = (inv - mu_ref[0, k]) / sg_ref[0, k]
        rows.append(jnp.exp(-0.5 * t * t) * cut)
    ht = jnp.dot(w_ref[...], gt, preferred_element_type=jnp.float32)
    acc0 = jnp.zeros((NF, BB), jnp.float32)
    acc1 = jnp.zeros((NF, BB), jnp.float32)
    for k in range(ND):
        hk = rows[k] * ht[k * NF:(k + 1) * NF, :]
        if k % 2 == 0:
            acc0 = acc0 + hk
        else:
            acc1 = acc1 + hk
    z_ref[...] = jnp.transpose(acc0 + acc1)


TD = 400


def _tc_out_body(p_ref, x_ref, w_ref, b_ref, o_ref):
    s = jnp.dot(x_ref[...], w_ref[...], preferred_element_type=jnp.float32)
    o_ref[...] = p_ref[0] + p_ref[1] + s + b_ref[...]


def kernel(in_features, pair_first, pair_second, dist_pairs, mu, sigma,
           int_weights, self_W, self_b):
    ps = pair_second.astype(jnp.int32)
    pf = pair_first.astype(jnp.int32)
    x = in_features.astype(jnp.float32)

    mesh = plsc.VectorSubcoreMesh(core_axis_name="c", subcore_axis_name="s")

    gather = pl.kernel(
        _sc_gather_body,
        out_type=jax.ShapeDtypeStruct((E, NF), jnp.float32),
        mesh=mesh,
        scratch_types=[
            pltpu.VMEM((CHUNK,), jnp.int32),
            pltpu.VMEM((CHUNK, NF), jnp.float32),
            pltpu.SemaphoreType.DMA,
        ],
    )
    g = gather(x, ps)

    wm = int_weights.reshape(ND * NF, NF)
    dist4 = dist_pairs.astype(jnp.float32).reshape(NB, 1, BB)
    mu2 = mu.astype(jnp.float32).reshape(1, ND)
    sg2 = sigma.astype(jnp.float32).reshape(1, ND)
    z = pl.pallas_call(
        _tc_z_body,
        grid=(NB,),
        in_specs=[
            pl.BlockSpec((BB, NF), lambda b: (b, 0)),
            pl.BlockSpec((1, 1, BB), lambda b: (b, 0, 0)),
            pl.BlockSpec((ND * NF, NF), lambda b: (0, 0)),
            pl.BlockSpec(memory_space=pltpu.SMEM),
            pl.BlockSpec(memory_space=pltpu.SMEM),
        ],
        out_specs=pl.BlockSpec((BB, NF), lambda b: (b, 0)),
        out_shape=jax.ShapeDtypeStruct((E, NF), jnp.float32),
    )(g, dist4, wm, mu2, sg2)

    scatter = pl.kernel(
        _sc_scatter_body,
        out_type=jax.ShapeDtypeStruct((2 * N, NF), jnp.float32),
        mesh=mesh,
        scratch_types=[
            pltpu.VMEM((CHUNK,), jnp.int32),
            pltpu.VMEM((CHUNK, NF), jnp.float32),
            pltpu.VMEM_SHARED((N, NF), jnp.float32),
        ],
    )
    partial = scatter(z, pf).reshape(2, N, NF)

    swt = jnp.transpose(self_W, (1, 0)).astype(jnp.float32)
    b2 = self_b.astype(jnp.float32).reshape(1, NF)
    out = pl.pallas_call(
        _tc_out_body,
        grid=(N // TD,),
        in_specs=[
            pl.BlockSpec((2, TD, NF), lambda b: (0, b, 0)),
            pl.BlockSpec((TD, NF), lambda b: (b, 0)),
            pl.BlockSpec((NF, NF), lambda b: (0, 0)),
            pl.BlockSpec((1, NF), lambda b: (0, 0)),
        ],
        out_specs=pl.BlockSpec((TD, NF), lambda b: (b, 0)),
        out_shape=jax.ShapeDtypeStruct((N, NF), jnp.float32),
    )(partial, x, swt, b2)
    return out


# speedup vs baseline: 2.2655x; 1.0893x over previous
import functools

import jax
import jax.numpy as jnp
from jax import lax
from jax.experimental import pallas as pl
from jax.experimental.pallas import tpu as pltpu
from jax.experimental.pallas import tpu_sc as plsc

N = 10000
E = 320000
NF = 128
ND = 20
HARD_CUTOFF = 6.5

NW = 32
CHUNK = 128
NCHUNKS = E // CHUNK
STEPS = (NCHUNKS + NW - 1) // NW

BB = 512
NB = E // BB

NFULL = N // CHUNK
NREM = N - NFULL * CHUNK


GG = 4
GSTEPS = (STEPS + GG - 1) // GG


def _sc_gather_body(x_hbm, ps_hbm, g_hbm,
                    i0, i1, i2, i3, r0, r1, r2, r3, sem_g, sem_w):
    wid = lax.axis_index("c") * 16 + lax.axis_index("s")
    idxs = [i0, i1, i2, i3]
    rows = [r0, r1, r2, r3]

    def grp(jj, carry):
        cs = [(jj * GG + b) * NW + wid for b in range(GG)]
        for b in range(GG):
            @pl.when(cs[b] < NCHUNKS)
            def _(b=b):
                pltpu.sync_copy(ps_hbm.at[pl.ds(cs[b] * CHUNK, CHUNK)], idxs[b])
        for b in range(GG):
            @pl.when(cs[b] < NCHUNKS)
            def _(b=b):
                pltpu.async_copy(x_hbm.at[idxs[b]], rows[b], sem_g)
        for b in range(GG):
            @pl.when(cs[b] < NCHUNKS)
            def _(b=b):
                pltpu.make_async_copy(x_hbm.at[idxs[b]], rows[b], sem_g).wait()
        for b in range(GG):
            @pl.when(cs[b] < NCHUNKS)
            def _(b=b):
                pltpu.async_copy(
                    rows[b], g_hbm.at[pl.ds(cs[b] * CHUNK, CHUNK)], sem_w)
        for b in range(GG):
            @pl.when(cs[b] < NCHUNKS)
            def _(b=b):
                pltpu.make_async_copy(
                    rows[b], g_hbm.at[pl.ds(cs[b] * CHUNK, CHUNK)], sem_w).wait()
        return carry

    lax.fori_loop(0, GSTEPS, grp, 0)


def _sc_scatter_body(z_hbm, pf_hbm, out_hbm, idx_v, idx_v2, rows_v, rows_v2,
                     acc_sh, sem_z, sem_s):
    cid = lax.axis_index("c")
    sid = lax.axis_index("s")
    wid = cid * 16 + sid

    zeros16 = jnp.zeros((16,), jnp.float32)

    def zstep(i, carry):
        r = i // (NF // 16)
        col = (i % (NF // 16)) * 16
        rows_v[r, pl.ds(col, 16)] = zeros16
        return carry

    lax.fori_loop(0, CHUNK * (NF // 16), zstep, 0)

    for i in range((NFULL + 15) // 16):
        blk = sid + i * 16

        @pl.when(blk < NFULL)
        def _():
            pltpu.sync_copy(rows_v, acc_sh.at[pl.ds(blk * CHUNK, CHUNK)])

    @pl.when(sid == 0)
    def _():
        pltpu.sync_copy(rows_v.at[pl.ds(0, NREM)],
                        acc_sh.at[pl.ds(NFULL * CHUNK, NREM)])

    plsc.subcore_barrier()

    idxs = [idx_v, idx_v2]
    rows = [rows_v, rows_v2]

    def grp(jj, carry):
        cs = [(jj * 2 + b) * NW + wid for b in range(2)]
        for b in range(2):
            @pl.when(cs[b] < NCHUNKS)
            def _(b=b):
                pltpu.sync_copy(pf_hbm.at[pl.ds(cs[b] * CHUNK, CHUNK)], idxs[b])
                pltpu.async_copy(
                    z_hbm.at[pl.ds(cs[b] * CHUNK, CHUNK)], rows[b], sem_z)
        for b in range(2):
            @pl.when(cs[b] < NCHUNKS)
            def _(b=b):
                pltpu.make_async_copy(
                    z_hbm.at[pl.ds(cs[b] * CHUNK, CHUNK)], rows[b], sem_z).wait()
        for b in range(2):
            @pl.when(cs[b] < NCHUNKS)
            def _(b=b):
                pltpu.async_copy(rows[b], acc_sh.at[idxs[b]], sem_s, add=True)
        for b in range(2):
            @pl.when(cs[b] < NCHUNKS)
            def _(b=b):
                pltpu.make_async_copy(
                    rows[b], acc_sh.at[idxs[b]], sem_s).wait()
        return carry

    lax.fori_loop(0, (STEPS + 1) // 2, grp, 0)
    plsc.subcore_barrier()

    for i in range((NFULL + 15) // 16):
        blk = sid + i * 16

        @pl.when(blk < NFULL)
        def _():
            pltpu.sync_copy(acc_sh.at[pl.ds(blk * CHUNK, CHUNK)], rows_v)
            pltpu.sync_copy(rows_v, out_hbm.at[pl.ds(cid * N + blk * CHUNK, CHUNK)])

    @pl.when(sid == 0)
    def _():
        pltpu.sync_copy(acc_sh.at[pl.ds(NFULL * CHUNK, NREM)],
                        rows_v.at[pl.ds(0, NREM)])
        pltpu.sync_copy(rows_v.at[pl.ds(0, NREM)],
                        out_hbm.at[pl.ds(cid * N + NFULL * CHUNK, NREM)])


def _tc_z_body(g_ref, d_ref, w_ref, mu_ref, sg_ref, z_ref):
    gt = jnp.transpose(g_ref[...])
    d = d_ref[0]
    inv = 1.0 / d
    cut = jnp.where(
        d < HARD_CUTOFF,
        jnp.cos(d * (jnp.pi / (2.0 * HARD_CUTOFF))) ** 2,
        0.0,
    )
    rows = []
    for k in range(ND):
        t = (inv - mu_ref[0, k]) / sg_ref[0, k]
        rows.append(jnp.exp(-0.5 * t * t) * cut)
    ht = jnp.dot(w_ref[...], gt, preferred_element_type=jnp.float32)
    acc0 = jnp.zeros((NF, BB), jnp.float32)
    acc1 = jnp.zeros((NF, BB), jnp.float32)
    for k in range(ND):
        hk = rows[k] * ht[k * NF:(k + 1) * NF, :]
        if k % 2 == 0:
            acc0 = acc0 + hk
        else:
            acc1 = acc1 + hk
    z_ref[...] = jnp.transpose(acc0 + acc1)


TD = 400


def _tc_out_body(p_ref, x_ref, w_ref, b_ref, o_ref):
    s = jnp.dot(x_ref[...], w_ref[...], preferred_element_type=jnp.float32)
    o_ref[...] = p_ref[0] + p_ref[1] + s + b_ref[...]


def kernel(in_features, pair_first, pair_second, dist_pairs, mu, sigma,
           int_weights, self_W, self_b):
    ps = pair_second.astype(jnp.int32)
    pf = pair_first.astype(jnp.int32)
    x = in_features.astype(jnp.float32)

    mesh = plsc.VectorSubcoreMesh(core_axis_name="c", subcore_axis_name="s")

    gather = pl.kernel(
        _sc_gather_body,
        out_type=jax.ShapeDtypeStruct((E, NF), jnp.float32),
        mesh=mesh,
        scratch_types=(
            [pltpu.VMEM((CHUNK,), jnp.int32)] * GG
            + [pltpu.VMEM((CHUNK, NF), jnp.float32)] * GG
            + [pltpu.SemaphoreType.DMA, pltpu.SemaphoreType.DMA]
        ),
    )
    g = gather(x, ps)

    wm = int_weights.reshape(ND * NF, NF)
    dist4 = dist_pairs.astype(jnp.float32).reshape(NB, 1, BB)
    mu2 = mu.astype(jnp.float32).reshape(1, ND)
    sg2 = sigma.astype(jnp.float32).reshape(1, ND)
    z = pl.pallas_call(
        _tc_z_body,
        grid=(NB,),
        in_specs=[
            pl.BlockSpec((BB, NF), lambda b: (b, 0)),
            pl.BlockSpec((1, 1, BB), lambda b: (b, 0, 0)),
            pl.BlockSpec((ND * NF, NF), lambda b: (0, 0)),
            pl.BlockSpec(memory_space=pltpu.SMEM),
            pl.BlockSpec(memory_space=pltpu.SMEM),
        ],
        out_specs=pl.BlockSpec((BB, NF), lambda b: (b, 0)),
        out_shape=jax.ShapeDtypeStruct((E, NF), jnp.float32),
    )(g, dist4, wm, mu2, sg2)

    scatter = pl.kernel(
        _sc_scatter_body,
        out_type=jax.ShapeDtypeStruct((2 * N, NF), jnp.float32),
        mesh=mesh,
        scratch_types=[
            pltpu.VMEM((CHUNK,), jnp.int32),
            pltpu.VMEM((CHUNK,), jnp.int32),
            pltpu.VMEM((CHUNK, NF), jnp.float32),
            pltpu.VMEM((CHUNK, NF), jnp.float32),
            pltpu.VMEM_SHARED((N, NF), jnp.float32),
            pltpu.SemaphoreType.DMA,
            pltpu.SemaphoreType.DMA,
        ],
    )
    partial = scatter(z, pf).reshape(2, N, NF)

    swt = jnp.transpose(self_W, (1, 0)).astype(jnp.float32)
    b2 = self_b.astype(jnp.float32).reshape(1, NF)
    out = pl.pallas_call(
        _tc_out_body,
        grid=(N // TD,),
        in_specs=[
            pl.BlockSpec((2, TD, NF), lambda b: (0, b, 0)),
            pl.BlockSpec((TD, NF), lambda b: (b, 0)),
            pl.BlockSpec((NF, NF), lambda b: (0, 0)),
            pl.BlockSpec((1, NF), lambda b: (0, 0)),
        ],
        out_specs=pl.BlockSpec((TD, NF), lambda b: (b, 0)),
        out_shape=jax.ShapeDtypeStruct((N, NF), jnp.float32),
    )(partial, x, swt, b2)
    return out


# speedup vs baseline: 2.7353x; 1.2074x over previous
import functools

import jax
import jax.numpy as jnp
from jax import lax
from jax.experimental import pallas as pl
from jax.experimental.pallas import tpu as pltpu
from jax.experimental.pallas import tpu_sc as plsc

N = 10000
E = 320000
NF = 128
ND = 20
HARD_CUTOFF = 6.5

NW = 32
CHUNK = 128
PARTS = 2
EP = E // PARTS
NCHUNKS = EP // CHUNK
STEPS = (NCHUNKS + NW - 1) // NW

BB = 640
NB = EP // BB

NFULL = N // CHUNK
NREM = N - NFULL * CHUNK


GG = 4
GSTEPS = (STEPS + GG - 1) // GG


def _sc_gather_body(x_hbm, ps_hbm, g_hbm,
                    i0, i1, i2, i3, r0, r1, r2, r3, sem_g, sem_w):
    wid = lax.axis_index("c") * 16 + lax.axis_index("s")
    idxs = [i0, i1, i2, i3]
    rows = [r0, r1, r2, r3]

    def grp(jj, carry):
        cs = [(jj * GG + b) * NW + wid for b in range(GG)]
        for b in range(GG):
            @pl.when(cs[b] < NCHUNKS)
            def _(b=b):
                pltpu.sync_copy(ps_hbm.at[pl.ds(cs[b] * CHUNK, CHUNK)], idxs[b])
        for b in range(GG):
            @pl.when(cs[b] < NCHUNKS)
            def _(b=b):
                pltpu.async_copy(x_hbm.at[idxs[b]], rows[b], sem_g)
        for b in range(GG):
            @pl.when(cs[b] < NCHUNKS)
            def _(b=b):
                pltpu.make_async_copy(x_hbm.at[idxs[b]], rows[b], sem_g).wait()
        for b in range(GG):
            @pl.when(cs[b] < NCHUNKS)
            def _(b=b):
                pltpu.async_copy(
                    rows[b], g_hbm.at[pl.ds(cs[b] * CHUNK, CHUNK)], sem_w)
        for b in range(GG):
            @pl.when(cs[b] < NCHUNKS)
            def _(b=b):
                pltpu.make_async_copy(
                    rows[b], g_hbm.at[pl.ds(cs[b] * CHUNK, CHUNK)], sem_w).wait()
        return carry

    lax.fori_loop(0, GSTEPS, grp, 0)


def _sc_scatter_body(z_hbm, pf_hbm, out_hbm, idx_v, idx_v2, rows_v, rows_v2,
                     acc_sh, sem_z, sem_s):
    cid = lax.axis_index("c")
    sid = lax.axis_index("s")
    wid = cid * 16 + sid

    zeros16 = jnp.zeros((16,), jnp.float32)

    def zstep(i, carry):
        r = i // (NF // 16)
        col = (i % (NF // 16)) * 16
        rows_v[r, pl.ds(col, 16)] = zeros16
        return carry

    lax.fori_loop(0, CHUNK * (NF // 16), zstep, 0)

    for i in range((NFULL + 15) // 16):
        blk = sid + i * 16

        @pl.when(blk < NFULL)
        def _():
            pltpu.sync_copy(rows_v, acc_sh.at[pl.ds(blk * CHUNK, CHUNK)])

    @pl.when(sid == 0)
    def _():
        pltpu.sync_copy(rows_v.at[pl.ds(0, NREM)],
                        acc_sh.at[pl.ds(NFULL * CHUNK, NREM)])

    plsc.subcore_barrier()

    idxs = [idx_v, idx_v2]
    rows = [rows_v, rows_v2]

    def grp(jj, carry):
        cs = [(jj * 2 + b) * NW + wid for b in range(2)]
        for b in range(2):
            @pl.when(cs[b] < NCHUNKS)
            def _(b=b):
                pltpu.sync_copy(pf_hbm.at[pl.ds(cs[b] * CHUNK, CHUNK)], idxs[b])
                pltpu.async_copy(
                    z_hbm.at[pl.ds(cs[b] * CHUNK, CHUNK)], rows[b], sem_z)
        for b in range(2):
            @pl.when(cs[b] < NCHUNKS)
            def _(b=b):
                pltpu.make_async_copy(
                    z_hbm.at[pl.ds(cs[b] * CHUNK, CHUNK)], rows[b], sem_z).wait()
        for b in range(2):
            @pl.when(cs[b] < NCHUNKS)
            def _(b=b):
                pltpu.async_copy(rows[b], acc_sh.at[idxs[b]], sem_s, add=True)
        for b in range(2):
            @pl.when(cs[b] < NCHUNKS)
            def _(b=b):
                pltpu.make_async_copy(
                    rows[b], acc_sh.at[idxs[b]], sem_s).wait()
        return carry

    lax.fori_loop(0, (STEPS + 1) // 2, grp, 0)
    plsc.subcore_barrier()

    for i in range((NFULL + 15) // 16):
        blk = sid + i * 16

        @pl.when(blk < NFULL)
        def _():
            pltpu.sync_copy(acc_sh.at[pl.ds(blk * CHUNK, CHUNK)], rows_v)
            pltpu.sync_copy(rows_v, out_hbm.at[pl.ds(cid * N + blk * CHUNK, CHUNK)])

    @pl.when(sid == 0)
    def _():
        pltpu.sync_copy(acc_sh.at[pl.ds(NFULL * CHUNK, NREM)],
                        rows_v.at[pl.ds(0, NREM)])
        pltpu.sync_copy(rows_v.at[pl.ds(0, NREM)],
                        out_hbm.at[pl.ds(cid * N + NFULL * CHUNK, NREM)])


def _tc_z_body(g_ref, d_ref, w_ref, mu_ref, sg_ref, z_ref):
    gt = jnp.transpose(g_ref[...])
    d = d_ref[0]
    inv = 1.0 / d
    cut = jnp.where(
        d < HARD_CUTOFF,
        jnp.cos(d * (jnp.pi / (2.0 * HARD_CUTOFF))) ** 2,
        0.0,
    )
    rows = []
    for k in range(ND):
        t = (inv - mu_ref[0, k]) / sg_ref[0, k]
        rows.append(jnp.exp(-0.5 * t * t) * cut)
    ht = jnp.dot(w_ref[...], gt, preferred_element_type=jnp.float32)
    acc0 = jnp.zeros((NF, BB), jnp.float32)
    acc1 = jnp.zeros((NF, BB), jnp.float32)
    for k in range(ND):
        hk = rows[k] * ht[k * NF:(k + 1) * NF, :]
        if k % 2 == 0:
            acc0 = acc0 + hk
        else:
            acc1 = acc1 + hk
    z_ref[...] = jnp.transpose(acc0 + acc1)


TD = 400


def _tc_out_body(p1_ref, p2_ref, x_ref, w_ref, b_ref, o_ref):
    s = jnp.dot(x_ref[...], w_ref[...], preferred_element_type=jnp.float32)
    o_ref[...] = ((p1_ref[0] + p1_ref[1]) + (p2_ref[0] + p2_ref[1])
                  + s + b_ref[...])


def kernel(in_features, pair_first, pair_second, dist_pairs, mu, sigma,
           int_weights, self_W, self_b):
    ps = pair_second.astype(jnp.int32)
    pf = pair_first.astype(jnp.int32)
    x = in_features.astype(jnp.float32)

    mesh = plsc.VectorSubcoreMesh(core_axis_name="c", subcore_axis_name="s")

    gather = pl.kernel(
        _sc_gather_body,
        out_type=jax.ShapeDtypeStruct((EP, NF), jnp.float32),
        mesh=mesh,
        scratch_types=(
            [pltpu.VMEM((CHUNK,), jnp.int32)] * GG
            + [pltpu.VMEM((CHUNK, NF), jnp.float32)] * GG
            + [pltpu.SemaphoreType.DMA, pltpu.SemaphoreType.DMA]
        ),
    )

    wm = int_weights.reshape(ND * NF, NF)
    mu2 = mu.astype(jnp.float32).reshape(1, ND)
    sg2 = sigma.astype(jnp.float32).reshape(1, ND)

    def tc_z(g_part, dist_part):
        return pl.pallas_call(
            _tc_z_body,
            grid=(NB,),
            in_specs=[
                pl.BlockSpec((BB, NF), lambda b: (b, 0)),
                pl.BlockSpec((1, 1, BB), lambda b: (b, 0, 0)),
                pl.BlockSpec((ND * NF, NF), lambda b: (0, 0)),
                pl.BlockSpec(memory_space=pltpu.SMEM),
                pl.BlockSpec(memory_space=pltpu.SMEM),
            ],
            out_specs=pl.BlockSpec((BB, NF), lambda b: (b, 0)),
            out_shape=jax.ShapeDtypeStruct((EP, NF), jnp.float32),
        )(g_part, dist_part.reshape(NB, 1, BB), wm, mu2, sg2)

    scatter = pl.kernel(
        _sc_scatter_body,
        out_type=jax.ShapeDtypeStruct((2 * N, NF), jnp.float32),
        mesh=mesh,
        scratch_types=[
            pltpu.VMEM((CHUNK,), jnp.int32),
            pltpu.VMEM((CHUNK,), jnp.int32),
            pltpu.VMEM((CHUNK, NF), jnp.float32),
            pltpu.VMEM((CHUNK, NF), jnp.float32),
            pltpu.VMEM_SHARED((N, NF), jnp.float32),
            pltpu.SemaphoreType.DMA,
            pltpu.SemaphoreType.DMA,
        ],
    )

    dist = dist_pairs.astype(jnp.float32)
    g1 = gather(x, ps[:EP])
    g2 = gather(x, ps[EP:])
    z1 = tc_z(g1, dist[:EP])
    p1 = scatter(z1, pf[:EP]).reshape(2, N, NF)
    z2 = tc_z(g2, dist[EP:])
    p2 = scatter(z2, pf[EP:]).reshape(2, N, NF)

    swt = jnp.transpose(self_W, (1, 0)).astype(jnp.float32)
    b2 = self_b.astype(jnp.float32).reshape(1, NF)
    out = pl.pallas_call(
        _tc_out_body,
        grid=(N // TD,),
        in_specs=[
            pl.BlockSpec((2, TD, NF), lambda b: (0, b, 0)),
            pl.BlockSpec((2, TD, NF), lambda b: (0, b, 0)),
            pl.BlockSpec((TD, NF), lambda b: (b, 0)),
            pl.BlockSpec((NF, NF), lambda b: (0, 0)),
            pl.BlockSpec((1, NF), lambda b: (0, 0)),
        ],
        out_specs=pl.BlockSpec((TD, NF), lambda b: (b, 0)),
        out_shape=jax.ShapeDtypeStruct((N, NF), jnp.float32),
    )(p1, p2, x, swt, b2)
    return out


# speedup vs baseline: 2.9247x; 1.0692x over previous
import functools

import jax
import jax.numpy as jnp
from jax import lax
from jax.experimental import pallas as pl
from jax.experimental.pallas import tpu as pltpu
from jax.experimental.pallas import tpu_sc as plsc

N = 10000
E = 320000
NF = 128
ND = 20
HARD_CUTOFF = 6.5

NW = 32
CHUNK = 128
PARTS = 4
EP = E // PARTS
NCHUNKS = EP // CHUNK
STEPS = (NCHUNKS + NW - 1) // NW

BB = 640
NB = EP // BB

NFULL = N // CHUNK
NREM = N - NFULL * CHUNK


GG = 4
GSTEPS = (STEPS + GG - 1) // GG


def _sc_gather_body(x_hbm, ps_hbm, g_hbm,
                    i0, i1, i2, i3, r0, r1, r2, r3, sem_g, sem_w):
    wid = lax.axis_index("c") * 16 + lax.axis_index("s")
    idxs = [i0, i1, i2, i3]
    rows = [r0, r1, r2, r3]

    def grp(jj, carry):
        cs = [(jj * GG + b) * NW + wid for b in range(GG)]
        for b in range(GG):
            @pl.when(cs[b] < NCHUNKS)
            def _(b=b):
                pltpu.sync_copy(ps_hbm.at[pl.ds(cs[b] * CHUNK, CHUNK)], idxs[b])
        for b in range(GG):
            @pl.when(cs[b] < NCHUNKS)
            def _(b=b):
                pltpu.async_copy(x_hbm.at[idxs[b]], rows[b], sem_g)
        for b in range(GG):
            @pl.when(cs[b] < NCHUNKS)
            def _(b=b):
                pltpu.make_async_copy(x_hbm.at[idxs[b]], rows[b], sem_g).wait()
        for b in range(GG):
            @pl.when(cs[b] < NCHUNKS)
            def _(b=b):
                pltpu.async_copy(
                    rows[b], g_hbm.at[pl.ds(cs[b] * CHUNK, CHUNK)], sem_w)
        for b in range(GG):
            @pl.when(cs[b] < NCHUNKS)
            def _(b=b):
                pltpu.make_async_copy(
                    rows[b], g_hbm.at[pl.ds(cs[b] * CHUNK, CHUNK)], sem_w).wait()
        return carry

    lax.fori_loop(0, GSTEPS, grp, 0)


def _sc_scatter_body(z_hbm, pf_hbm, out_hbm, idx_v, idx_v2, rows_v, rows_v2,
                     acc_sh, sem_z, sem_s):
    cid = lax.axis_index("c")
    sid = lax.axis_index("s")
    wid = cid * 16 + sid

    zeros16 = jnp.zeros((16,), jnp.float32)

    def zstep(i, carry):
        r = i // (NF // 16)
        col = (i % (NF // 16)) * 16
        rows_v[r, pl.ds(col, 16)] = zeros16
        return carry

    lax.fori_loop(0, CHUNK * (NF // 16), zstep, 0)

    for i in range((NFULL + 15) // 16):
        blk = sid + i * 16

        @pl.when(blk < NFULL)
        def _():
            pltpu.sync_copy(rows_v, acc_sh.at[pl.ds(blk * CHUNK, CHUNK)])

    @pl.when(sid == 0)
    def _():
        pltpu.sync_copy(rows_v.at[pl.ds(0, NREM)],
                        acc_sh.at[pl.ds(NFULL * CHUNK, NREM)])

    plsc.subcore_barrier()

    idxs = [idx_v, idx_v2]
    rows = [rows_v, rows_v2]

    def grp(jj, carry):
        cs = [(jj * 2 + b) * NW + wid for b in range(2)]
        for b in range(2):
            @pl.when(cs[b] < NCHUNKS)
            def _(b=b):
                pltpu.sync_copy(pf_hbm.at[pl.ds(cs[b] * CHUNK, CHUNK)], idxs[b])
                pltpu.async_copy(
                    z_hbm.at[pl.ds(cs[b] * CHUNK, CHUNK)], rows[b], sem_z)
        for b in range(2):
            @pl.when(cs[b] < NCHUNKS)
            def _(b=b):
                pltpu.make_async_copy(
                    z_hbm.at[pl.ds(cs[b] * CHUNK, CHUNK)], rows[b], sem_z).wait()
        for b in range(2):
            @pl.when(cs[b] < NCHUNKS)
            def _(b=b):
                pltpu.async_copy(rows[b], acc_sh.at[idxs[b]], sem_s, add=True)
        for b in range(2):
            @pl.when(cs[b] < NCHUNKS)
            def _(b=b):
                pltpu.make_async_copy(
                    rows[b], acc_sh.at[idxs[b]], sem_s).wait()
        return carry

    lax.fori_loop(0, (STEPS + 1) // 2, grp, 0)
    plsc.subcore_barrier()

    for i in range((NFULL + 15) // 16):
        blk = sid + i * 16

        @pl.when(blk < NFULL)
        def _():
            pltpu.sync_copy(acc_sh.at[pl.ds(blk * CHUNK, CHUNK)], rows_v)
            pltpu.sync_copy(rows_v, out_hbm.at[pl.ds(cid * N + blk * CHUNK, CHUNK)])

    @pl.when(sid == 0)
    def _():
        pltpu.sync_copy(acc_sh.at[pl.ds(NFULL * CHUNK, NREM)],
                        rows_v.at[pl.ds(0, NREM)])
        pltpu.sync_copy(rows_v.at[pl.ds(0, NREM)],
                        out_hbm.at[pl.ds(cid * N + NFULL * CHUNK, NREM)])


def _tc_z_body(g_ref, d_ref, w_ref, mu_ref, sg_ref, z_ref):
    gt = jnp.transpose(g_ref[...])
    d = d_ref[0]
    inv = 1.0 / d
    cut = jnp.where(
        d < HARD_CUTOFF,
        jnp.cos(d * (jnp.pi / (2.0 * HARD_CUTOFF))) ** 2,
        0.0,
    )
    rows = []
    for k in range(ND):
        t = (inv - mu_ref[0, k]) / sg_ref[0, k]
        rows.append(jnp.exp(-0.5 * t * t) * cut)
    ht = jnp.dot(w_ref[...], gt, preferred_element_type=jnp.float32)
    acc0 = jnp.zeros((NF, BB), jnp.float32)
    acc1 = jnp.zeros((NF, BB), jnp.float32)
    for k in range(ND):
        hk = rows[k] * ht[k * NF:(k + 1) * NF, :]
        if k % 2 == 0:
            acc0 = acc0 + hk
        else:
            acc1 = acc1 + hk
    z_ref[...] = jnp.transpose(acc0 + acc1)


TD = 400


def _tc_out_body(p1_ref, p2_ref, x_ref, w_ref, b_ref, o_ref):
    s = jnp.dot(x_ref[...], w_ref[...], preferred_element_type=jnp.float32)
    o_ref[...] = ((p1_ref[0] + p1_ref[1]) + (p2_ref[0] + p2_ref[1])
                  + s + b_ref[...])


def kernel(in_features, pair_first, pair_second, dist_pairs, mu, sigma,
           int_weights, self_W, self_b):
    ps = pair_second.astype(jnp.int32)
    pf = pair_first.astype(jnp.int32)
    x = in_features.astype(jnp.float32)

    mesh = plsc.VectorSubcoreMesh(core_axis_name="c", subcore_axis_name="s")

    gather = pl.kernel(
        _sc_gather_body,
        out_type=jax.ShapeDtypeStruct((EP, NF), jnp.float32),
        mesh=mesh,
        scratch_types=(
            [pltpu.VMEM((CHUNK,), jnp.int32)] * GG
            + [pltpu.VMEM((CHUNK, NF), jnp.float32)] * GG
            + [pltpu.SemaphoreType.DMA, pltpu.SemaphoreType.DMA]
        ),
    )

    wm = int_weights.reshape(ND * NF, NF)
    mu2 = mu.astype(jnp.float32).reshape(1, ND)
    sg2 = sigma.astype(jnp.float32).reshape(1, ND)

    def tc_z(g_part, dist_part):
        return pl.pallas_call(
            _tc_z_body,
            grid=(NB,),
            in_specs=[
                pl.BlockSpec((BB, NF), lambda b: (b, 0)),
                pl.BlockSpec((1, 1, BB), lambda b: (b, 0, 0)),
                pl.BlockSpec((ND * NF, NF), lambda b: (0, 0)),
                pl.BlockSpec(memory_space=pltpu.SMEM),
                pl.BlockSpec(memory_space=pltpu.SMEM),
            ],
            out_specs=pl.BlockSpec((BB, NF), lambda b: (b, 0)),
            out_shape=jax.ShapeDtypeStruct((EP, NF), jnp.float32),
        )(g_part, dist_part.reshape(NB, 1, BB), wm, mu2, sg2)

    scatter = pl.kernel(
        _sc_scatter_body,
        out_type=jax.ShapeDtypeStruct((2 * N, NF), jnp.float32),
        mesh=mesh,
        scratch_types=[
            pltpu.VMEM((CHUNK,), jnp.int32),
            pltpu.VMEM((CHUNK,), jnp.int32),
            pltpu.VMEM((CHUNK, NF), jnp.float32),
            pltpu.VMEM((CHUNK, NF), jnp.float32),
            pltpu.VMEM_SHARED((N, NF), jnp.float32),
            pltpu.SemaphoreType.DMA,
            pltpu.SemaphoreType.DMA,
        ],
    )

    dist = dist_pairs.astype(jnp.float32)
    gs = [gather(x, ps[i * EP:(i + 1) * EP]) for i in range(PARTS)]
    zs = [tc_z(gs[i], dist[i * EP:(i + 1) * EP]) for i in range(PARTS)]
    pps = [scatter(zs[i], pf[i * EP:(i + 1) * EP]).reshape(2, N, NF)
           for i in range(PARTS)]
    p1 = pps[0] + pps[1]
    p2 = pps[2] + pps[3]

    swt = jnp.transpose(self_W, (1, 0)).astype(jnp.float32)
    b2 = self_b.astype(jnp.float32).reshape(1, NF)
    out = pl.pallas_call(
        _tc_out_body,
        grid=(N // TD,),
        in_specs=[
            pl.BlockSpec((2, TD, NF), lambda b: (0, b, 0)),
            pl.BlockSpec((2, TD, NF), lambda b: (0, b, 0)),
            pl.BlockSpec((TD, NF), lambda b: (b, 0)),
            pl.BlockSpec((NF, NF), lambda b: (0, 0)),
            pl.BlockSpec((1, NF), lambda b: (0, 0)),
        ],
        out_specs=pl.BlockSpec((TD, NF), lambda b: (b, 0)),
        out_shape=jax.ShapeDtypeStruct((N, NF), jnp.float32),
    )(p1, p2, x, swt, b2)
    return out


# speedup vs baseline: 2.9644x; 1.0136x over previous
import functools

import jax
import jax.numpy as jnp
from jax import lax
from jax.experimental import pallas as pl
from jax.experimental.pallas import tpu as pltpu
from jax.experimental.pallas import tpu_sc as plsc

N = 10000
E = 320000
NF = 128
ND = 20
HARD_CUTOFF = 6.5

NW = 32
CHUNK = 128
PARTS = 4
EP = E // PARTS
NCHUNKS = EP // CHUNK
STEPS = (NCHUNKS + NW - 1) // NW

BB = 640
NB = EP // BB

NFULL = N // CHUNK
NREM = N - NFULL * CHUNK


GG = 4
GSTEPS = (STEPS + GG - 1) // GG


def _sc_gather_body(x_hbm, ps_hbm, g_hbm,
                    i0, i1, i2, i3, r0, r1, r2, r3, sem_g, sem_w):
    wid = lax.axis_index("c") * 16 + lax.axis_index("s")
    idxs = [i0, i1, i2, i3]
    rows = [r0, r1, r2, r3]

    def grp(jj, carry):
        cs = [(jj * GG + b) * NW + wid for b in range(GG)]
        for b in range(GG):
            @pl.when(cs[b] < NCHUNKS)
            def _(b=b):
                pltpu.sync_copy(ps_hbm.at[pl.ds(cs[b] * CHUNK, CHUNK)], idxs[b])
        for b in range(GG):
            @pl.when(cs[b] < NCHUNKS)
            def _(b=b):
                pltpu.async_copy(x_hbm.at[idxs[b]], rows[b], sem_g)
        for b in range(GG):
            @pl.when(cs[b] < NCHUNKS)
            def _(b=b):
                pltpu.make_async_copy(x_hbm.at[idxs[b]], rows[b], sem_g).wait()
        for b in range(GG):
            @pl.when(cs[b] < NCHUNKS)
            def _(b=b):
                pltpu.async_copy(
                    rows[b], g_hbm.at[pl.ds(cs[b] * CHUNK, CHUNK)], sem_w)
        for b in range(GG):
            @pl.when(cs[b] < NCHUNKS)
            def _(b=b):
                pltpu.make_async_copy(
                    rows[b], g_hbm.at[pl.ds(cs[b] * CHUNK, CHUNK)], sem_w).wait()
        return carry

    lax.fori_loop(0, GSTEPS, grp, 0)


def _sc_scatter_body(z_hbm, pf_hbm, out_hbm, idx_v, idx_v2, rows_v, rows_v2,
                     acc_sh, sem_z, sem_s):
    cid = lax.axis_index("c")
    sid = lax.axis_index("s")
    wid = cid * 16 + sid

    zeros16 = jnp.zeros((16,), jnp.float32)

    def zstep(i, carry):
        r = i // (NF // 16)
        col = (i % (NF // 16)) * 16
        rows_v[r, pl.ds(col, 16)] = zeros16
        return carry

    lax.fori_loop(0, CHUNK * (NF // 16), zstep, 0)

    for i in range((NFULL + 15) // 16):
        blk = sid + i * 16

        @pl.when(blk < NFULL)
        def _():
            pltpu.sync_copy(rows_v, acc_sh.at[pl.ds(blk * CHUNK, CHUNK)])

    @pl.when(sid == 0)
    def _():
        pltpu.sync_copy(rows_v.at[pl.ds(0, NREM)],
                        acc_sh.at[pl.ds(NFULL * CHUNK, NREM)])

    plsc.subcore_barrier()

    idxs = [idx_v, idx_v2]
    rows = [rows_v, rows_v2]

    def grp(jj, carry):
        cs = [(jj * 2 + b) * NW + wid for b in range(2)]
        for b in range(2):
            @pl.when(cs[b] < NCHUNKS)
            def _(b=b):
                pltpu.sync_copy(pf_hbm.at[pl.ds(cs[b] * CHUNK, CHUNK)], idxs[b])
                pltpu.async_copy(
                    z_hbm.at[pl.ds(cs[b] * CHUNK, CHUNK)], rows[b], sem_z)
        for b in range(2):
            @pl.when(cs[b] < NCHUNKS)
            def _(b=b):
                pltpu.make_async_copy(
                    z_hbm.at[pl.ds(cs[b] * CHUNK, CHUNK)], rows[b], sem_z).wait()
        for b in range(2):
            @pl.when(cs[b] < NCHUNKS)
            def _(b=b):
                pltpu.async_copy(rows[b], acc_sh.at[idxs[b]], sem_s, add=True)
        for b in range(2):
            @pl.when(cs[b] < NCHUNKS)
            def _(b=b):
                pltpu.make_async_copy(
                    rows[b], acc_sh.at[idxs[b]], sem_s).wait()
        return carry

    lax.fori_loop(0, (STEPS + 1) // 2, grp, 0)
    plsc.subcore_barrier()

    for i in range((NFULL + 15) // 16):
        blk = sid + i * 16

        @pl.when(blk < NFULL)
        def _():
            pltpu.sync_copy(acc_sh.at[pl.ds(blk * CHUNK, CHUNK)], rows_v)
            pltpu.sync_copy(rows_v, out_hbm.at[pl.ds(cid * N + blk * CHUNK, CHUNK)])

    @pl.when(sid == 0)
    def _():
        pltpu.sync_copy(acc_sh.at[pl.ds(NFULL * CHUNK, NREM)],
                        rows_v.at[pl.ds(0, NREM)])
        pltpu.sync_copy(rows_v.at[pl.ds(0, NREM)],
                        out_hbm.at[pl.ds(cid * N + NFULL * CHUNK, NREM)])


def _tc_z_body(g_ref, d_ref, w_ref, mu_ref, sg_ref, z_ref):
    gt = jnp.transpose(g_ref[...])
    d = d_ref[0]
    inv = 1.0 / d
    cut = jnp.where(
        d < HARD_CUTOFF,
        jnp.cos(d * (jnp.pi / (2.0 * HARD_CUTOFF))) ** 2,
        0.0,
    )
    rows = []
    for k in range(ND):
        t = (inv - mu_ref[0, k]) / sg_ref[0, k]
        rows.append(jnp.exp(-0.5 * t * t) * cut)
    ht = jnp.dot(w_ref[...], gt, preferred_element_type=jnp.float32)
    acc0 = jnp.zeros((NF, BB), jnp.float32)
    acc1 = jnp.zeros((NF, BB), jnp.float32)
    for k in range(ND):
        hk = rows[k] * ht[k * NF:(k + 1) * NF, :]
        if k % 2 == 0:
            acc0 = acc0 + hk
        else:
            acc1 = acc1 + hk
    z_ref[...] = jnp.transpose(acc0 + acc1)


TD = 400


def _tc_out_body(p1_ref, p2_ref, p3_ref, p4_ref, x_ref, w_ref, b_ref, o_ref):
    s = jnp.dot(x_ref[...], w_ref[...], preferred_element_type=jnp.float32)
    o_ref[...] = (((p1_ref[0] + p1_ref[1]) + (p2_ref[0] + p2_ref[1]))
                  + ((p3_ref[0] + p3_ref[1]) + (p4_ref[0] + p4_ref[1]))
                  + s + b_ref[...])


def kernel(in_features, pair_first, pair_second, dist_pairs, mu, sigma,
           int_weights, self_W, self_b):
    ps = pair_second.astype(jnp.int32)
    pf = pair_first.astype(jnp.int32)
    x = in_features.astype(jnp.float32)

    mesh = plsc.VectorSubcoreMesh(core_axis_name="c", subcore_axis_name="s")

    gather = pl.kernel(
        _sc_gather_body,
        out_type=jax.ShapeDtypeStruct((EP, NF), jnp.float32),
        mesh=mesh,
        scratch_types=(
            [pltpu.VMEM((CHUNK,), jnp.int32)] * GG
            + [pltpu.VMEM((CHUNK, NF), jnp.float32)] * GG
            + [pltpu.SemaphoreType.DMA, pltpu.SemaphoreType.DMA]
        ),
    )

    wm = int_weights.reshape(ND * NF, NF)
    mu2 = mu.astype(jnp.float32).reshape(1, ND)
    sg2 = sigma.astype(jnp.float32).reshape(1, ND)

    def tc_z(g_part, dist_part):
        return pl.pallas_call(
            _tc_z_body,
            grid=(NB,),
            in_specs=[
                pl.BlockSpec((BB, NF), lambda b: (b, 0)),
                pl.BlockSpec((1, 1, BB), lambda b: (b, 0, 0)),
                pl.BlockSpec((ND * NF, NF), lambda b: (0, 0)),
                pl.BlockSpec(memory_space=pltpu.SMEM),
                pl.BlockSpec(memory_space=pltpu.SMEM),
            ],
            out_specs=pl.BlockSpec((BB, NF), lambda b: (b, 0)),
            out_shape=jax.ShapeDtypeStruct((EP, NF), jnp.float32),
        )(g_part, dist_part.reshape(NB, 1, BB), wm, mu2, sg2)

    scatter = pl.kernel(
        _sc_scatter_body,
        out_type=jax.ShapeDtypeStruct((2 * N, NF), jnp.float32),
        mesh=mesh,
        scratch_types=[
            pltpu.VMEM((CHUNK,), jnp.int32),
            pltpu.VMEM((CHUNK,), jnp.int32),
            pltpu.VMEM((CHUNK, NF), jnp.float32),
            pltpu.VMEM((CHUNK, NF), jnp.float32),
            pltpu.VMEM_SHARED((N, NF), jnp.float32),
            pltpu.SemaphoreType.DMA,
            pltpu.SemaphoreType.DMA,
        ],
    )

    dist = dist_pairs.astype(jnp.float32)
    gs = [gather(x, ps[i * EP:(i + 1) * EP]) for i in range(PARTS)]
    zs = [tc_z(gs[i], dist[i * EP:(i + 1) * EP]) for i in range(PARTS)]
    pps = [scatter(zs[i], pf[i * EP:(i + 1) * EP]).reshape(2, N, NF)
           for i in range(PARTS)]

    swt = jnp.transpose(self_W, (1, 0)).astype(jnp.float32)
    b2 = self_b.astype(jnp.float32).reshape(1, NF)
    out = pl.pallas_call(
        _tc_out_body,
        grid=(N // TD,),
        in_specs=[
            pl.BlockSpec((2, TD, NF), lambda b: (0, b, 0)),
            pl.BlockSpec((2, TD, NF), lambda b: (0, b, 0)),
            pl.BlockSpec((2, TD, NF), lambda b: (0, b, 0)),
            pl.BlockSpec((2, TD, NF), lambda b: (0, b, 0)),
            pl.BlockSpec((TD, NF), lambda b: (b, 0)),
            pl.BlockSpec((NF, NF), lambda b: (0, 0)),
            pl.BlockSpec((1, NF), lambda b: (0, 0)),
        ],
        out_specs=pl.BlockSpec((TD, NF), lambda b: (b, 0)),
        out_shape=jax.ShapeDtypeStruct((N, NF), jnp.float32),
    )(pps[0], pps[1], pps[2], pps[3], x, swt, b2)
    return out


# speedup vs baseline: 3.2006x; 1.0797x over previous
import functools

import jax
import jax.numpy as jnp
from jax import lax
from jax.experimental import pallas as pl
from jax.experimental.pallas import tpu as pltpu
from jax.experimental.pallas import tpu_sc as plsc

N = 10000
E = 320000
NF = 128
ND = 20
HARD_CUTOFF = 6.5

NW = 32
CHUNK = 128
PARTS = 4
EP = E // PARTS
NCHUNKS = EP // CHUNK
STEPS = (NCHUNKS + NW - 1) // NW

BB = 640
NB = EP // BB

NFULL = N // CHUNK
NREM = N - NFULL * CHUNK


GG = 4
GSTEPS = (STEPS + GG - 1) // GG


def _sc_gather_body(x_hbm, ps_hbm, g_hbm,
                    i0, i1, i2, i3, r0, r1, r2, r3, sem_g, sem_w):
    wid = lax.axis_index("c") * 16 + lax.axis_index("s")
    idxs = [i0, i1, i2, i3]
    rows = [r0, r1, r2, r3]

    def grp(jj, carry):
        cs = [(jj * GG + b) * NW + wid for b in range(GG)]
        for b in range(GG):
            @pl.when(cs[b] < NCHUNKS)
            def _(b=b):
                pltpu.sync_copy(ps_hbm.at[pl.ds(cs[b] * CHUNK, CHUNK)], idxs[b])
        for b in range(GG):
            @pl.when(cs[b] < NCHUNKS)
            def _(b=b):
                pltpu.async_copy(x_hbm.at[idxs[b]], rows[b], sem_g)
        for b in range(GG):
            @pl.when(cs[b] < NCHUNKS)
            def _(b=b):
                pltpu.make_async_copy(x_hbm.at[idxs[b]], rows[b], sem_g).wait()
        for b in range(GG):
            @pl.when(cs[b] < NCHUNKS)
            def _(b=b):
                pltpu.async_copy(
                    rows[b], g_hbm.at[pl.ds(cs[b] * CHUNK, CHUNK)], sem_w)
        for b in range(GG):
            @pl.when(cs[b] < NCHUNKS)
            def _(b=b):
                pltpu.make_async_copy(
                    rows[b], g_hbm.at[pl.ds(cs[b] * CHUNK, CHUNK)], sem_w).wait()
        return carry

    lax.fori_loop(0, GSTEPS, grp, 0)


def _sc_scatter_body(z_hbm, pf_hbm, out_hbm, idx_v, idx_v2, rows_v, rows_v2,
                     acc_sh, sem_z, sem_s):
    cid = lax.axis_index("c")
    sid = lax.axis_index("s")
    wid = cid * 16 + sid

    zeros16 = jnp.zeros((16,), jnp.float32)

    def zstep(i, carry):
        r = i // (NF // 16)
        col = (i % (NF // 16)) * 16
        rows_v[r, pl.ds(col, 16)] = zeros16
        return carry

    lax.fori_loop(0, CHUNK * (NF // 16), zstep, 0)

    for i in range((NFULL + 15) // 16):
        blk = sid + i * 16

        @pl.when(blk < NFULL)
        def _():
            pltpu.sync_copy(rows_v, acc_sh.at[pl.ds(blk * CHUNK, CHUNK)])

    @pl.when(sid == 0)
    def _():
        pltpu.sync_copy(rows_v.at[pl.ds(0, NREM)],
                        acc_sh.at[pl.ds(NFULL * CHUNK, NREM)])

    plsc.subcore_barrier()

    idxs = [idx_v, idx_v2]
    rows = [rows_v, rows_v2]

    def grp(jj, carry):
        cs = [(jj * 2 + b) * NW + wid for b in range(2)]
        for b in range(2):
            @pl.when(cs[b] < NCHUNKS)
            def _(b=b):
                pltpu.sync_copy(pf_hbm.at[pl.ds(cs[b] * CHUNK, CHUNK)], idxs[b])
                pltpu.async_copy(
                    z_hbm.at[pl.ds(cs[b] * CHUNK, CHUNK)], rows[b], sem_z)
        for b in range(2):
            @pl.when(cs[b] < NCHUNKS)
            def _(b=b):
                pltpu.make_async_copy(
                    z_hbm.at[pl.ds(cs[b] * CHUNK, CHUNK)], rows[b], sem_z).wait()
        for b in range(2):
            @pl.when(cs[b] < NCHUNKS)
            def _(b=b):
                pltpu.async_copy(rows[b], acc_sh.at[idxs[b]], sem_s, add=True)
        for b in range(2):
            @pl.when(cs[b] < NCHUNKS)
            def _(b=b):
                pltpu.make_async_copy(
                    rows[b], acc_sh.at[idxs[b]], sem_s).wait()
        return carry

    lax.fori_loop(0, (STEPS + 1) // 2, grp, 0)
    plsc.subcore_barrier()

    for i in range((NFULL + 15) // 16):
        blk = sid + i * 16

        @pl.when(blk < NFULL)
        def _():
            pltpu.sync_copy(acc_sh.at[pl.ds(blk * CHUNK, CHUNK)], rows_v)
            pltpu.sync_copy(rows_v, out_hbm.at[pl.ds(cid * N + blk * CHUNK, CHUNK)])

    @pl.when(sid == 0)
    def _():
        pltpu.sync_copy(acc_sh.at[pl.ds(NFULL * CHUNK, NREM)],
                        rows_v.at[pl.ds(0, NREM)])
        pltpu.sync_copy(rows_v.at[pl.ds(0, NREM)],
                        out_hbm.at[pl.ds(cid * N + NFULL * CHUNK, NREM)])


def _tc_z_body(g_ref, d_ref, w_ref, mu_ref, sg_ref, z_ref):
    gt = jnp.transpose(g_ref[...])
    d = d_ref[0]
    inv = 1.0 / d
    cut = jnp.where(
        d < HARD_CUTOFF,
        jnp.cos(d * (jnp.pi / (2.0 * HARD_CUTOFF))) ** 2,
        0.0,
    )
    rows = []
    for k in range(ND):
        t = (inv - mu_ref[0, k]) / sg_ref[0, k]
        rows.append(jnp.exp(-0.5 * t * t) * cut)
    kr = jnp.concatenate([rows[k] * gt for k in range(ND)], axis=0)
    zt = jnp.dot(w_ref[...], kr, preferred_element_type=jnp.float32)
    z_ref[...] = jnp.transpose(zt)


TD = 400


def _tc_out_body(p1_ref, p2_ref, p3_ref, p4_ref, x_ref, w_ref, b_ref, o_ref):
    s = jnp.dot(x_ref[...], w_ref[...], preferred_element_type=jnp.float32)
    o_ref[...] = (((p1_ref[0] + p1_ref[1]) + (p2_ref[0] + p2_ref[1]))
                  + ((p3_ref[0] + p3_ref[1]) + (p4_ref[0] + p4_ref[1]))
                  + s + b_ref[...])


def kernel(in_features, pair_first, pair_second, dist_pairs, mu, sigma,
           int_weights, self_W, self_b):
    ps = pair_second.astype(jnp.int32)
    pf = pair_first.astype(jnp.int32)
    x = in_features.astype(jnp.float32)

    mesh = plsc.VectorSubcoreMesh(core_axis_name="c", subcore_axis_name="s")

    gather = pl.kernel(
        _sc_gather_body,
        out_type=jax.ShapeDtypeStruct((EP, NF), jnp.float32),
        mesh=mesh,
        scratch_types=(
            [pltpu.VMEM((CHUNK,), jnp.int32)] * GG
            + [pltpu.VMEM((CHUNK, NF), jnp.float32)] * GG
            + [pltpu.SemaphoreType.DMA, pltpu.SemaphoreType.DMA]
        ),
    )

    wm = jnp.transpose(int_weights, (1, 0, 2)).reshape(NF, ND * NF)
    mu2 = mu.astype(jnp.float32).reshape(1, ND)
    sg2 = sigma.astype(jnp.float32).reshape(1, ND)

    def tc_z(g_part, dist_part):
        return pl.pallas_call(
            _tc_z_body,
            grid=(NB,),
            in_specs=[
                pl.BlockSpec((BB, NF), lambda b: (b, 0)),
                pl.BlockSpec((1, 1, BB), lambda b: (b, 0, 0)),
                pl.BlockSpec((NF, ND * NF), lambda b: (0, 0)),
                pl.BlockSpec(memory_space=pltpu.SMEM),
                pl.BlockSpec(memory_space=pltpu.SMEM),
            ],
            out_specs=pl.BlockSpec((BB, NF), lambda b: (b, 0)),
            out_shape=jax.ShapeDtypeStruct((EP, NF), jnp.float32),
        )(g_part, dist_part.reshape(NB, 1, BB), wm, mu2, sg2)

    scatter = pl.kernel(
        _sc_scatter_body,
        out_type=jax.ShapeDtypeStruct((2 * N, NF), jnp.float32),
        mesh=mesh,
        scratch_types=[
            pltpu.VMEM((CHUNK,), jnp.int32),
            pltpu.VMEM((CHUNK,), jnp.int32),
            pltpu.VMEM((CHUNK, NF), jnp.float32),
            pltpu.VMEM((CHUNK, NF), jnp.float32),
            pltpu.VMEM_SHARED((N, NF), jnp.float32),
            pltpu.SemaphoreType.DMA,
            pltpu.SemaphoreType.DMA,
        ],
    )

    dist = dist_pairs.astype(jnp.float32)
    gs = [gather(x, ps[i * EP:(i + 1) * EP]) for i in range(PARTS)]
    zs = [tc_z(gs[i], dist[i * EP:(i + 1) * EP]) for i in range(PARTS)]
    pps = [scatter(zs[i], pf[i * EP:(i + 1) * EP]).reshape(2, N, NF)
           for i in range(PARTS)]

    swt = jnp.transpose(self_W, (1, 0)).astype(jnp.float32)
    b2 = self_b.astype(jnp.float32).reshape(1, NF)
    out = pl.pallas_call(
        _tc_out_body,
        grid=(N // TD,),
        in_specs=[
            pl.BlockSpec((2, TD, NF), lambda b: (0, b, 0)),
            pl.BlockSpec((2, TD, NF), lambda b: (0, b, 0)),
            pl.BlockSpec((2, TD, NF), lambda b: (0, b, 0)),
            pl.BlockSpec((2, TD, NF), lambda b: (0, b, 0)),
            pl.BlockSpec((TD, NF), lambda b: (b, 0)),
            pl.BlockSpec((NF, NF), lambda b: (0, 0)),
            pl.BlockSpec((1, NF), lambda b: (0, 0)),
        ],
        out_specs=pl.BlockSpec((TD, NF), lambda b: (b, 0)),
        out_shape=jax.ShapeDtypeStruct((N, NF), jnp.float32),
    )(pps[0], pps[1], pps[2], pps[3], x, swt, b2)
    return out


# speedup vs baseline: 3.2372x; 1.0115x over previous
import functools

import jax
import jax.numpy as jnp
from jax import lax
from jax.experimental import pallas as pl
from jax.experimental.pallas import tpu as pltpu
from jax.experimental.pallas import tpu_sc as plsc

N = 10000
E = 320000
NF = 128
ND = 20
HARD_CUTOFF = 6.5

NW = 32
CHUNK = 128
PARTS = 5
EP = E // PARTS
NCHUNKS = EP // CHUNK
STEPS = (NCHUNKS + NW - 1) // NW

BB = 640
NB = EP // BB

NFULL = N // CHUNK
NREM = N - NFULL * CHUNK


GG = 4
GSTEPS = (STEPS + GG - 1) // GG


def _sc_gather_body(x_hbm, ps_hbm, g_hbm,
                    i0, i1, i2, i3, r0, r1, r2, r3, sem_g, sem_w):
    wid = lax.axis_index("c") * 16 + lax.axis_index("s")
    idxs = [i0, i1, i2, i3]
    rows = [r0, r1, r2, r3]

    def grp(jj, carry):
        cs = [(jj * GG + b) * NW + wid for b in range(GG)]
        for b in range(GG):
            @pl.when(cs[b] < NCHUNKS)
            def _(b=b):
                pltpu.sync_copy(ps_hbm.at[pl.ds(cs[b] * CHUNK, CHUNK)], idxs[b])
        for b in range(GG):
            @pl.when(cs[b] < NCHUNKS)
            def _(b=b):
                pltpu.async_copy(x_hbm.at[idxs[b]], rows[b], sem_g)
        for b in range(GG):
            @pl.when(cs[b] < NCHUNKS)
            def _(b=b):
                pltpu.make_async_copy(x_hbm.at[idxs[b]], rows[b], sem_g).wait()
        for b in range(GG):
            @pl.when(cs[b] < NCHUNKS)
            def _(b=b):
                pltpu.async_copy(
                    rows[b], g_hbm.at[pl.ds(cs[b] * CHUNK, CHUNK)], sem_w)
        for b in range(GG):
            @pl.when(cs[b] < NCHUNKS)
            def _(b=b):
                pltpu.make_async_copy(
                    rows[b], g_hbm.at[pl.ds(cs[b] * CHUNK, CHUNK)], sem_w).wait()
        return carry

    lax.fori_loop(0, GSTEPS, grp, 0)


def _sc_scatter_body(z_hbm, pf_hbm, out_hbm, idx_v, idx_v2, rows_v, rows_v2,
                     acc_sh, sem_z, sem_s):
    cid = lax.axis_index("c")
    sid = lax.axis_index("s")
    wid = cid * 16 + sid

    zeros16 = jnp.zeros((16,), jnp.float32)

    def zstep(i, carry):
        r = i // (NF // 16)
        col = (i % (NF // 16)) * 16
        rows_v[r, pl.ds(col, 16)] = zeros16
        return carry

    lax.fori_loop(0, CHUNK * (NF // 16), zstep, 0)

    for i in range((NFULL + 15) // 16):
        blk = sid + i * 16

        @pl.when(blk < NFULL)
        def _():
            pltpu.sync_copy(rows_v, acc_sh.at[pl.ds(blk * CHUNK, CHUNK)])

    @pl.when(sid == 0)
    def _():
        pltpu.sync_copy(rows_v.at[pl.ds(0, NREM)],
                        acc_sh.at[pl.ds(NFULL * CHUNK, NREM)])

    plsc.subcore_barrier()

    idxs = [idx_v, idx_v2]
    rows = [rows_v, rows_v2]

    def grp(jj, carry):
        cs = [(jj * 2 + b) * NW + wid for b in range(2)]
        for b in range(2):
            @pl.when(cs[b] < NCHUNKS)
            def _(b=b):
                pltpu.sync_copy(pf_hbm.at[pl.ds(cs[b] * CHUNK, CHUNK)], idxs[b])
                pltpu.async_copy(
                    z_hbm.at[pl.ds(cs[b] * CHUNK, CHUNK)], rows[b], sem_z)
        for b in range(2):
            @pl.when(cs[b] < NCHUNKS)
            def _(b=b):
                pltpu.make_async_copy(
                    z_hbm.at[pl.ds(cs[b] * CHUNK, CHUNK)], rows[b], sem_z).wait()
        for b in range(2):
            @pl.when(cs[b] < NCHUNKS)
            def _(b=b):
                pltpu.async_copy(rows[b], acc_sh.at[idxs[b]], sem_s, add=True)
        for b in range(2):
            @pl.when(cs[b] < NCHUNKS)
            def _(b=b):
                pltpu.make_async_copy(
                    rows[b], acc_sh.at[idxs[b]], sem_s).wait()
        return carry

    lax.fori_loop(0, (STEPS + 1) // 2, grp, 0)
    plsc.subcore_barrier()

    for i in range((NFULL + 15) // 16):
        blk = sid + i * 16

        @pl.when(blk < NFULL)
        def _():
            pltpu.sync_copy(acc_sh.at[pl.ds(blk * CHUNK, CHUNK)], rows_v)
            pltpu.sync_copy(rows_v, out_hbm.at[pl.ds(cid * N + blk * CHUNK, CHUNK)])

    @pl.when(sid == 0)
    def _():
        pltpu.sync_copy(acc_sh.at[pl.ds(NFULL * CHUNK, NREM)],
                        rows_v.at[pl.ds(0, NREM)])
        pltpu.sync_copy(rows_v.at[pl.ds(0, NREM)],
                        out_hbm.at[pl.ds(cid * N + NFULL * CHUNK, NREM)])


def _tc_z_body(g_ref, d_ref, w_ref, mu_ref, sg_ref, z_ref):
    gt = jnp.transpose(g_ref[...])
    d = d_ref[0]
    inv = 1.0 / d
    cut = jnp.where(
        d < HARD_CUTOFF,
        jnp.cos(d * (jnp.pi / (2.0 * HARD_CUTOFF))) ** 2,
        0.0,
    )
    rows = []
    for k in range(ND):
        t = (inv - mu_ref[0, k]) / sg_ref[0, k]
        rows.append(jnp.exp(-0.5 * t * t) * cut)
    kr = jnp.concatenate([rows[k] * gt for k in range(ND)], axis=0)
    zt = jnp.dot(w_ref[...], kr, preferred_element_type=jnp.float32)
    z_ref[...] = jnp.transpose(zt)


TD = 400


def _tc_out_body(p1_ref, p2_ref, p3_ref, p4_ref, p5_ref, x_ref, w_ref, b_ref,
                 o_ref):
    s = jnp.dot(x_ref[...], w_ref[...], preferred_element_type=jnp.float32)
    o_ref[...] = (((p1_ref[0] + p1_ref[1]) + (p2_ref[0] + p2_ref[1]))
                  + ((p3_ref[0] + p3_ref[1]) + (p4_ref[0] + p4_ref[1]))
                  + (p5_ref[0] + p5_ref[1]) + s + b_ref[...])


def kernel(in_features, pair_first, pair_second, dist_pairs, mu, sigma,
           int_weights, self_W, self_b):
    ps = pair_second.astype(jnp.int32)
    pf = pair_first.astype(jnp.int32)
    x = in_features.astype(jnp.float32)

    mesh = plsc.VectorSubcoreMesh(core_axis_name="c", subcore_axis_name="s")

    gather = pl.kernel(
        _sc_gather_body,
        out_type=jax.ShapeDtypeStruct((EP, NF), jnp.float32),
        mesh=mesh,
        scratch_types=(
            [pltpu.VMEM((CHUNK,), jnp.int32)] * GG
            + [pltpu.VMEM((CHUNK, NF), jnp.float32)] * GG
            + [pltpu.SemaphoreType.DMA, pltpu.SemaphoreType.DMA]
        ),
    )

    wm = jnp.transpose(int_weights, (1, 0, 2)).reshape(NF, ND * NF)
    mu2 = mu.astype(jnp.float32).reshape(1, ND)
    sg2 = sigma.astype(jnp.float32).reshape(1, ND)

    def tc_z(g_part, dist_part):
        return pl.pallas_call(
            _tc_z_body,
            grid=(NB,),
            in_specs=[
                pl.BlockSpec((BB, NF), lambda b: (b, 0)),
                pl.BlockSpec((1, 1, BB), lambda b: (b, 0, 0)),
                pl.BlockSpec((NF, ND * NF), lambda b: (0, 0)),
                pl.BlockSpec(memory_space=pltpu.SMEM),
                pl.BlockSpec(memory_space=pltpu.SMEM),
            ],
            out_specs=pl.BlockSpec((BB, NF), lambda b: (b, 0)),
            out_shape=jax.ShapeDtypeStruct((EP, NF), jnp.float32),
        )(g_part, dist_part.reshape(NB, 1, BB), wm, mu2, sg2)

    scatter = pl.kernel(
        _sc_scatter_body,
        out_type=jax.ShapeDtypeStruct((2 * N, NF), jnp.float32),
        mesh=mesh,
        scratch_types=[
            pltpu.VMEM((CHUNK,), jnp.int32),
            pltpu.VMEM((CHUNK,), jnp.int32),
            pltpu.VMEM((CHUNK, NF), jnp.float32),
            pltpu.VMEM((CHUNK, NF), jnp.float32),
            pltpu.VMEM_SHARED((N, NF), jnp.float32),
            pltpu.SemaphoreType.DMA,
            pltpu.SemaphoreType.DMA,
        ],
    )

    dist = dist_pairs.astype(jnp.float32)
    gs = [gather(x, ps[i * EP:(i + 1) * EP]) for i in range(PARTS)]
    zs = [tc_z(gs[i], dist[i * EP:(i + 1) * EP]) for i in range(PARTS)]
    pps = [scatter(zs[i], pf[i * EP:(i + 1) * EP]).reshape(2, N, NF)
           for i in range(PARTS)]

    swt = jnp.transpose(self_W, (1, 0)).astype(jnp.float32)
    b2 = self_b.astype(jnp.float32).reshape(1, NF)
    out = pl.pallas_call(
        _tc_out_body,
        grid=(N // TD,),
        in_specs=[
            pl.BlockSpec((2, TD, NF), lambda b: (0, b, 0)),
            pl.BlockSpec((2, TD, NF), lambda b: (0, b, 0)),
            pl.BlockSpec((2, TD, NF), lambda b: (0, b, 0)),
            pl.BlockSpec((2, TD, NF), lambda b: (0, b, 0)),
            pl.BlockSpec((2, TD, NF), lambda b: (0, b, 0)),
            pl.BlockSpec((TD, NF), lambda b: (b, 0)),
            pl.BlockSpec((NF, NF), lambda b: (0, 0)),
            pl.BlockSpec((1, NF), lambda b: (0, 0)),
        ],
        out_specs=pl.BlockSpec((TD, NF), lambda b: (b, 0)),
        out_shape=jax.ShapeDtypeStruct((N, NF), jnp.float32),
    )(pps[0], pps[1], pps[2], pps[3], pps[4], x, swt, b2)
    return out


# speedup vs baseline: 3.2605x; 1.0072x over previous
import functools

import jax
import jax.numpy as jnp
from jax import lax
from jax.experimental import pallas as pl
from jax.experimental.pallas import tpu as pltpu
from jax.experimental.pallas import tpu_sc as plsc

N = 10000
E = 320000
NF = 128
ND = 20
HARD_CUTOFF = 6.5

NW = 32
CHUNK = 128
PARTS = 5
EP = E // PARTS
NCHUNKS = EP // CHUNK
STEPS = (NCHUNKS + NW - 1) // NW

BB = 640
NB = EP // BB

NFULL = N // CHUNK
NREM = N - NFULL * CHUNK


GG = 6
GSTEPS = (STEPS + GG - 1) // GG


def _sc_gather_body(x_hbm, ps_hbm, g_hbm,
                    i0, i1, i2, i3, i4, i5, r0, r1, r2, r3, r4, r5,
                    sem_i, sem_g, sem_w):
    wid = lax.axis_index("c") * 16 + lax.axis_index("s")
    idxs = [i0, i1, i2, i3, i4, i5]
    rows = [r0, r1, r2, r3, r4, r5]

    def grp(jj, carry):
        cs = [(jj * GG + b) * NW + wid for b in range(GG)]
        for b in range(GG):
            @pl.when(cs[b] < NCHUNKS)
            def _(b=b):
                pltpu.async_copy(ps_hbm.at[pl.ds(cs[b] * CHUNK, CHUNK)], idxs[b],
                                 sem_i)
        for b in range(GG):
            @pl.when(cs[b] < NCHUNKS)
            def _(b=b):
                pltpu.make_async_copy(ps_hbm.at[pl.ds(cs[b] * CHUNK, CHUNK)],
                                      idxs[b], sem_i).wait()
        for b in range(GG):
            @pl.when(cs[b] < NCHUNKS)
            def _(b=b):
                pltpu.async_copy(x_hbm.at[idxs[b]], rows[b], sem_g)
        for b in range(GG):
            @pl.when(cs[b] < NCHUNKS)
            def _(b=b):
                pltpu.make_async_copy(x_hbm.at[idxs[b]], rows[b], sem_g).wait()
        for b in range(GG):
            @pl.when(cs[b] < NCHUNKS)
            def _(b=b):
                pltpu.async_copy(
                    rows[b], g_hbm.at[pl.ds(cs[b] * CHUNK, CHUNK)], sem_w)
        for b in range(GG):
            @pl.when(cs[b] < NCHUNKS)
            def _(b=b):
                pltpu.make_async_copy(
                    rows[b], g_hbm.at[pl.ds(cs[b] * CHUNK, CHUNK)], sem_w).wait()
        return carry

    lax.fori_loop(0, GSTEPS, grp, 0)


def _sc_scatter_body(z_hbm, pf_hbm, out_hbm, idx_v, idx_v2, rows_v, rows_v2,
                     acc_sh, sem_z, sem_s):
    cid = lax.axis_index("c")
    sid = lax.axis_index("s")
    wid = cid * 16 + sid

    zeros16 = jnp.zeros((16,), jnp.float32)

    def zstep(i, carry):
        r = i // (NF // 16)
        col = (i % (NF // 16)) * 16
        rows_v[r, pl.ds(col, 16)] = zeros16
        return carry

    lax.fori_loop(0, CHUNK * (NF // 16), zstep, 0)

    for i in range((NFULL + 15) // 16):
        blk = sid + i * 16

        @pl.when(blk < NFULL)
        def _():
            pltpu.sync_copy(rows_v, acc_sh.at[pl.ds(blk * CHUNK, CHUNK)])

    @pl.when(sid == 0)
    def _():
        pltpu.sync_copy(rows_v.at[pl.ds(0, NREM)],
                        acc_sh.at[pl.ds(NFULL * CHUNK, NREM)])

    plsc.subcore_barrier()

    idxs = [idx_v, idx_v2]
    rows = [rows_v, rows_v2]

    def grp(jj, carry):
        cs = [(jj * 2 + b) * NW + wid for b in range(2)]
        for b in range(2):
            @pl.when(cs[b] < NCHUNKS)
            def _(b=b):
                pltpu.sync_copy(pf_hbm.at[pl.ds(cs[b] * CHUNK, CHUNK)], idxs[b])
                pltpu.async_copy(
                    z_hbm.at[pl.ds(cs[b] * CHUNK, CHUNK)], rows[b], sem_z)
        for b in range(2):
            @pl.when(cs[b] < NCHUNKS)
            def _(b=b):
                pltpu.make_async_copy(
                    z_hbm.at[pl.ds(cs[b] * CHUNK, CHUNK)], rows[b], sem_z).wait()
        for b in range(2):
            @pl.when(cs[b] < NCHUNKS)
            def _(b=b):
                pltpu.async_copy(rows[b], acc_sh.at[idxs[b]], sem_s, add=True)
        for b in range(2):
            @pl.when(cs[b] < NCHUNKS)
            def _(b=b):
                pltpu.make_async_copy(
                    rows[b], acc_sh.at[idxs[b]], sem_s).wait()
        return carry

    lax.fori_loop(0, (STEPS + 1) // 2, grp, 0)
    plsc.subcore_barrier()

    for i in range((NFULL + 15) // 16):
        blk = sid + i * 16

        @pl.when(blk < NFULL)
        def _():
            pltpu.sync_copy(acc_sh.at[pl.ds(blk * CHUNK, CHUNK)], rows_v)
            pltpu.sync_copy(rows_v, out_hbm.at[pl.ds(cid * N + blk * CHUNK, CHUNK)])

    @pl.when(sid == 0)
    def _():
        pltpu.sync_copy(acc_sh.at[pl.ds(NFULL * CHUNK, NREM)],
                        rows_v.at[pl.ds(0, NREM)])
        pltpu.sync_copy(rows_v.at[pl.ds(0, NREM)],
                        out_hbm.at[pl.ds(cid * N + NFULL * CHUNK, NREM)])


def _tc_z_body(g_ref, d_ref, w_ref, mu_ref, sg_ref, z_ref):
    gt = jnp.transpose(g_ref[...])
    d = d_ref[0]
    inv = 1.0 / d
    cut = jnp.where(
        d < HARD_CUTOFF,
        jnp.cos(d * (jnp.pi / (2.0 * HARD_CUTOFF))) ** 2,
        0.0,
    )
    rows = []
    for k in range(ND):
        t = (inv - mu_ref[0, k]) / sg_ref[0, k]
        rows.append(jnp.exp(-0.5 * t * t) * cut)
    kr = jnp.concatenate([rows[k] * gt for k in range(ND)], axis=0)
    zt = jnp.dot(w_ref[...], kr, preferred_element_type=jnp.float32)
    z_ref[...] = jnp.transpose(zt)


TD = 400


def _tc_out_body(p1_ref, p2_ref, p3_ref, p4_ref, p5_ref, x_ref, w_ref, b_ref,
                 o_ref):
    s = jnp.dot(x_ref[...], w_ref[...], preferred_element_type=jnp.float32)
    o_ref[...] = (((p1_ref[0] + p1_ref[1]) + (p2_ref[0] + p2_ref[1]))
                  + ((p3_ref[0] + p3_ref[1]) + (p4_ref[0] + p4_ref[1]))
                  + (p5_ref[0] + p5_ref[1]) + s + b_ref[...])


def kernel(in_features, pair_first, pair_second, dist_pairs, mu, sigma,
           int_weights, self_W, self_b):
    ps = pair_second.astype(jnp.int32)
    pf = pair_first.astype(jnp.int32)
    x = in_features.astype(jnp.float32)

    mesh = plsc.VectorSubcoreMesh(core_axis_name="c", subcore_axis_name="s")

    gather = pl.kernel(
        _sc_gather_body,
        out_type=jax.ShapeDtypeStruct((EP, NF), jnp.float32),
        mesh=mesh,
        scratch_types=(
            [pltpu.VMEM((CHUNK,), jnp.int32)] * GG
            + [pltpu.VMEM((CHUNK, NF), jnp.float32)] * GG
            + [pltpu.SemaphoreType.DMA] * 3
        ),
    )

    wm = jnp.transpose(int_weights, (1, 0, 2)).reshape(NF, ND * NF)
    mu2 = mu.astype(jnp.float32).reshape(1, ND)
    sg2 = sigma.astype(jnp.float32).reshape(1, ND)

    def tc_z(g_part, dist_part):
        return pl.pallas_call(
            _tc_z_body,
            grid=(NB,),
            in_specs=[
                pl.BlockSpec((BB, NF), lambda b: (b, 0)),
                pl.BlockSpec((1, 1, BB), lambda b: (b, 0, 0)),
                pl.BlockSpec((NF, ND * NF), lambda b: (0, 0)),
                pl.BlockSpec(memory_space=pltpu.SMEM),
                pl.BlockSpec(memory_space=pltpu.SMEM),
            ],
            out_specs=pl.BlockSpec((BB, NF), lambda b: (b, 0)),
            out_shape=jax.ShapeDtypeStruct((EP, NF), jnp.float32),
        )(g_part, dist_part.reshape(NB, 1, BB), wm, mu2, sg2)

    scatter = pl.kernel(
        _sc_scatter_body,
        out_type=jax.ShapeDtypeStruct((2 * N, NF), jnp.float32),
        mesh=mesh,
        scratch_types=[
            pltpu.VMEM((CHUNK,), jnp.int32),
            pltpu.VMEM((CHUNK,), jnp.int32),
            pltpu.VMEM((CHUNK, NF), jnp.float32),
            pltpu.VMEM((CHUNK, NF), jnp.float32),
            pltpu.VMEM_SHARED((N, NF), jnp.float32),
            pltpu.SemaphoreType.DMA,
            pltpu.SemaphoreType.DMA,
        ],
    )

    dist = dist_pairs.astype(jnp.float32)
    gs = [gather(x, ps[i * EP:(i + 1) * EP]) for i in range(PARTS)]
    zs = [tc_z(gs[i], dist[i * EP:(i + 1) * EP]) for i in range(PARTS)]
    pps = [scatter(zs[i], pf[i * EP:(i + 1) * EP]).reshape(2, N, NF)
           for i in range(PARTS)]

    swt = jnp.transpose(self_W, (1, 0)).astype(jnp.float32)
    b2 = self_b.astype(jnp.float32).reshape(1, NF)
    out = pl.pallas_call(
        _tc_out_body,
        grid=(N // TD,),
        in_specs=[
            pl.BlockSpec((2, TD, NF), lambda b: (0, b, 0)),
            pl.BlockSpec((2, TD, NF), lambda b: (0, b, 0)),
            pl.BlockSpec((2, TD, NF), lambda b: (0, b, 0)),
            pl.BlockSpec((2, TD, NF), lambda b: (0, b, 0)),
            pl.BlockSpec((2, TD, NF), lambda b: (0, b, 0)),
            pl.BlockSpec((TD, NF), lambda b: (b, 0)),
            pl.BlockSpec((NF, NF), lambda b: (0, 0)),
            pl.BlockSpec((1, NF), lambda b: (0, 0)),
        ],
        out_specs=pl.BlockSpec((TD, NF), lambda b: (b, 0)),
        out_shape=jax.ShapeDtypeStruct((N, NF), jnp.float32),
    )(pps[0], pps[1], pps[2], pps[3], pps[4], x, swt, b2)
    return out


# speedup vs baseline: 4.3281x; 1.3274x over previous
import functools

import jax
import jax.numpy as jnp
from jax import lax
from jax.experimental import pallas as pl
from jax.experimental.pallas import tpu as pltpu
from jax.experimental.pallas import tpu_sc as plsc

N = 10000
E = 320000
NF = 128
ND = 20
HARD_CUTOFF = 6.5

NW = 32
CHUNK = 128
PARTS = 5
EP = E // PARTS
NCHUNKS = EP // CHUNK
STEPS = (NCHUNKS + NW - 1) // NW

BB = 1600
NB = EP // BB

NFULL = N // CHUNK
NREM = N - NFULL * CHUNK


GG = 6
GSTEPS = (STEPS + GG - 1) // GG


def _sc_gather_body(x_hbm, ps_hbm, g_hbm,
                    i0, i1, i2, i3, i4, i5, r0, r1, r2, r3, r4, r5,
                    sem_i, sem_g, sem_w):
    wid = lax.axis_index("c") * 16 + lax.axis_index("s")
    idxs = [i0, i1, i2, i3, i4, i5]
    rows = [r0, r1, r2, r3, r4, r5]

    def grp(jj, carry):
        cs = [(jj * GG + b) * NW + wid for b in range(GG)]
        for b in range(GG):
            @pl.when(cs[b] < NCHUNKS)
            def _(b=b):
                pltpu.async_copy(ps_hbm.at[pl.ds(cs[b] * CHUNK, CHUNK)], idxs[b],
                                 sem_i)
        for b in range(GG):
            @pl.when(cs[b] < NCHUNKS)
            def _(b=b):
                pltpu.make_async_copy(ps_hbm.at[pl.ds(cs[b] * CHUNK, CHUNK)],
                                      idxs[b], sem_i).wait()
        for b in range(GG):
            @pl.when(cs[b] < NCHUNKS)
            def _(b=b):
                pltpu.async_copy(x_hbm.at[idxs[b]], rows[b], sem_g)
        for b in range(GG):
            @pl.when(cs[b] < NCHUNKS)
            def _(b=b):
                pltpu.make_async_copy(x_hbm.at[idxs[b]], rows[b], sem_g).wait()
        for b in range(GG):
            @pl.when(cs[b] < NCHUNKS)
            def _(b=b):
                pltpu.async_copy(
                    rows[b], g_hbm.at[pl.ds(cs[b] * CHUNK, CHUNK)], sem_w)
        for b in range(GG):
            @pl.when(cs[b] < NCHUNKS)
            def _(b=b):
                pltpu.make_async_copy(
                    rows[b], g_hbm.at[pl.ds(cs[b] * CHUNK, CHUNK)], sem_w).wait()
        return carry

    lax.fori_loop(0, GSTEPS, grp, 0)


def _sc_scatter_body(z_hbm, pf_hbm, out_hbm, idx_v, idx_v2, rows_v, rows_v2,
                     acc_sh, sem_z, sem_s):
    cid = lax.axis_index("c")
    sid = lax.axis_index("s")
    wid = cid * 16 + sid

    zeros16 = jnp.zeros((16,), jnp.float32)

    def zstep(i, carry):
        r = i // (NF // 16)
        col = (i % (NF // 16)) * 16
        rows_v[r, pl.ds(col, 16)] = zeros16
        return carry

    lax.fori_loop(0, CHUNK * (NF // 16), zstep, 0)

    for i in range((NFULL + 15) // 16):
        blk = sid + i * 16

        @pl.when(blk < NFULL)
        def _():
            pltpu.sync_copy(rows_v, acc_sh.at[pl.ds(blk * CHUNK, CHUNK)])

    @pl.when(sid == 0)
    def _():
        pltpu.sync_copy(rows_v.at[pl.ds(0, NREM)],
                        acc_sh.at[pl.ds(NFULL * CHUNK, NREM)])

    plsc.subcore_barrier()

    idxs = [idx_v, idx_v2]
    rows = [rows_v, rows_v2]

    def grp(jj, carry):
        cs = [(jj * 2 + b) * NW + wid for b in range(2)]
        for b in range(2):
            @pl.when(cs[b] < NCHUNKS)
            def _(b=b):
                pltpu.sync_copy(pf_hbm.at[pl.ds(cs[b] * CHUNK, CHUNK)], idxs[b])
                pltpu.async_copy(
                    z_hbm.at[pl.ds(cs[b] * CHUNK, CHUNK)], rows[b], sem_z)
        for b in range(2):
            @pl.when(cs[b] < NCHUNKS)
            def _(b=b):
                pltpu.make_async_copy(
                    z_hbm.at[pl.ds(cs[b] * CHUNK, CHUNK)], rows[b], sem_z).wait()
        for b in range(2):
            @pl.when(cs[b] < NCHUNKS)
            def _(b=b):
                pltpu.async_copy(rows[b], acc_sh.at[idxs[b]], sem_s, add=True)
        for b in range(2):
            @pl.when(cs[b] < NCHUNKS)
            def _(b=b):
                pltpu.make_async_copy(
                    rows[b], acc_sh.at[idxs[b]], sem_s).wait()
        return carry

    lax.fori_loop(0, (STEPS + 1) // 2, grp, 0)
    plsc.subcore_barrier()

    for i in range((NFULL + 15) // 16):
        blk = sid + i * 16

        @pl.when(blk < NFULL)
        def _():
            pltpu.sync_copy(acc_sh.at[pl.ds(blk * CHUNK, CHUNK)], rows_v)
            pltpu.sync_copy(rows_v, out_hbm.at[pl.ds(cid * N + blk * CHUNK, CHUNK)])

    @pl.when(sid == 0)
    def _():
        pltpu.sync_copy(acc_sh.at[pl.ds(NFULL * CHUNK, NREM)],
                        rows_v.at[pl.ds(0, NREM)])
        pltpu.sync_copy(rows_v.at[pl.ds(0, NREM)],
                        out_hbm.at[pl.ds(cid * N + NFULL * CHUNK, NREM)])


def _tc_z_body(g_ref, d_ref, w_ref, mu_ref, sg_ref, z_ref):
    gt = jnp.transpose(g_ref[...])
    d = d_ref[0]
    inv = 1.0 / d
    cut = jnp.where(
        d < HARD_CUTOFF,
        jnp.cos(d * (jnp.pi / (2.0 * HARD_CUTOFF))) ** 2,
        0.0,
    )
    rows = []
    for k in range(ND):
        t = (inv - mu_ref[0, k]) / sg_ref[0, k]
        rows.append(jnp.exp(-0.5 * t * t) * cut)
    kr = jnp.concatenate([rows[k] * gt for k in range(ND)], axis=0)
    zt = jnp.dot(w_ref[...], kr, preferred_element_type=jnp.float32)
    z_ref[...] = jnp.transpose(zt)


TD = 400


def _tc_out_body(p1_ref, p2_ref, p3_ref, p4_ref, p5_ref, x_ref, w_ref, b_ref,
                 o_ref):
    s = jnp.dot(x_ref[...], w_ref[...], preferred_element_type=jnp.float32)
    o_ref[...] = (((p1_ref[0] + p1_ref[1]) + (p2_ref[0] + p2_ref[1]))
                  + ((p3_ref[0] + p3_ref[1]) + (p4_ref[0] + p4_ref[1]))
                  + (p5_ref[0] + p5_ref[1]) + s + b_ref[...])


def kernel(in_features, pair_first, pair_second, dist_pairs, mu, sigma,
           int_weights, self_W, self_b):
    ps = pair_second.astype(jnp.int32)
    pf = pair_first.astype(jnp.int32)
    x = in_features.astype(jnp.float32)

    mesh = plsc.VectorSubcoreMesh(core_axis_name="c", subcore_axis_name="s")

    gather = pl.kernel(
        _sc_gather_body,
        out_type=jax.ShapeDtypeStruct((EP, NF), jnp.float32),
        mesh=mesh,
        scratch_types=(
            [pltpu.VMEM((CHUNK,), jnp.int32)] * GG
            + [pltpu.VMEM((CHUNK, NF), jnp.float32)] * GG
            + [pltpu.SemaphoreType.DMA] * 3
        ),
    )

    wm = jnp.transpose(int_weights, (1, 0, 2)).reshape(NF, ND * NF)
    mu2 = mu.astype(jnp.float32).reshape(1, ND)
    sg2 = sigma.astype(jnp.float32).reshape(1, ND)

    def tc_z(g_part, dist_part):
        return pl.pallas_call(
            _tc_z_body,
            grid=(NB,),
            in_specs=[
                pl.BlockSpec((BB, NF), lambda b: (b, 0)),
                pl.BlockSpec((1, 1, BB), lambda b: (b, 0, 0)),
                pl.BlockSpec((NF, ND * NF), lambda b: (0, 0)),
                pl.BlockSpec(memory_space=pltpu.SMEM),
                pl.BlockSpec(memory_space=pltpu.SMEM),
            ],
            out_specs=pl.BlockSpec((BB, NF), lambda b: (b, 0)),
            out_shape=jax.ShapeDtypeStruct((EP, NF), jnp.float32),
        )(g_part, dist_part.reshape(NB, 1, BB), wm, mu2, sg2)

    scatter = pl.kernel(
        _sc_scatter_body,
        out_type=jax.ShapeDtypeStruct((2 * N, NF), jnp.float32),
        mesh=mesh,
        scratch_types=[
            pltpu.VMEM((CHUNK,), jnp.int32),
            pltpu.VMEM((CHUNK,), jnp.int32),
            pltpu.VMEM((CHUNK, NF), jnp.float32),
            pltpu.VMEM((CHUNK, NF), jnp.float32),
            pltpu.VMEM_SHARED((N, NF), jnp.float32),
            pltpu.SemaphoreType.DMA,
            pltpu.SemaphoreType.DMA,
        ],
    )

    dist = dist_pairs.astype(jnp.float32)
    gs = [gather(x, ps[i * EP:(i + 1) * EP]) for i in range(PARTS)]
    zs = [tc_z(gs[i], dist[i * EP:(i + 1) * EP]) for i in range(PARTS)]
    pps = [scatter(zs[i], pf[i * EP:(i + 1) * EP]).reshape(2, N, NF)
           for i in range(PARTS)]

    swt = jnp.transpose(self_W, (1, 0)).astype(jnp.float32)
    b2 = self_b.astype(jnp.float32).reshape(1, NF)
    out = pl.pallas_call(
        _tc_out_body,
        grid=(N // TD,),
        in_specs=[
            pl.BlockSpec((2, TD, NF), lambda b: (0, b, 0)),
            pl.BlockSpec((2, TD, NF), lambda b: (0, b, 0)),
            pl.BlockSpec((2, TD, NF), lambda b: (0, b, 0)),
            pl.BlockSpec((2, TD, NF), lambda b: (0, b, 0)),
            pl.BlockSpec((2, TD, NF), lambda b: (0, b, 0)),
            pl.BlockSpec((TD, NF), lambda b: (b, 0)),
            pl.BlockSpec((NF, NF), lambda b: (0, 0)),
            pl.BlockSpec((1, NF), lambda b: (0, 0)),
        ],
        out_specs=pl.BlockSpec((TD, NF), lambda b: (b, 0)),
        out_shape=jax.ShapeDtypeStruct((N, NF), jnp.float32),
    )(pps[0], pps[1], pps[2], pps[3], pps[4], x, swt, b2)
    return out


# speedup vs baseline: 4.8383x; 1.1179x over previous
import functools

import jax
import jax.numpy as jnp
from jax import lax
from jax.experimental import pallas as pl
from jax.experimental.pallas import tpu as pltpu
from jax.experimental.pallas import tpu_sc as plsc

N = 10000
E = 320000
NF = 128
ND = 20
HARD_CUTOFF = 6.5

NW = 32
CHUNK = 128
PARTS = 5
EP = E // PARTS
NCHUNKS = EP // CHUNK
STEPS = (NCHUNKS + NW - 1) // NW

BB = 3200
NB = EP // BB

NFULL = N // CHUNK
NREM = N - NFULL * CHUNK


GG = 6
GSTEPS = (STEPS + GG - 1) // GG


def _sc_gather_body(x_hbm, ps_hbm, g_hbm,
                    i0, i1, i2, i3, i4, i5, r0, r1, r2, r3, r4, r5,
                    sem_i, sem_g, sem_w):
    wid = lax.axis_index("c") * 16 + lax.axis_index("s")
    idxs = [i0, i1, i2, i3, i4, i5]
    rows = [r0, r1, r2, r3, r4, r5]

    def grp(jj, carry):
        cs = [(jj * GG + b) * NW + wid for b in range(GG)]
        for b in range(GG):
            @pl.when(cs[b] < NCHUNKS)
            def _(b=b):
                pltpu.async_copy(ps_hbm.at[pl.ds(cs[b] * CHUNK, CHUNK)], idxs[b],
                                 sem_i)
        for b in range(GG):
            @pl.when(cs[b] < NCHUNKS)
            def _(b=b):
                pltpu.make_async_copy(ps_hbm.at[pl.ds(cs[b] * CHUNK, CHUNK)],
                                      idxs[b], sem_i).wait()
        for b in range(GG):
            @pl.when(cs[b] < NCHUNKS)
            def _(b=b):
                pltpu.async_copy(x_hbm.at[idxs[b]], rows[b], sem_g)
        for b in range(GG):
            @pl.when(cs[b] < NCHUNKS)
            def _(b=b):
                pltpu.make_async_copy(x_hbm.at[idxs[b]], rows[b], sem_g).wait()
        for b in range(GG):
            @pl.when(cs[b] < NCHUNKS)
            def _(b=b):
                pltpu.async_copy(
                    rows[b], g_hbm.at[pl.ds(cs[b] * CHUNK, CHUNK)], sem_w)
        for b in range(GG):
            @pl.when(cs[b] < NCHUNKS)
            def _(b=b):
                pltpu.make_async_copy(
                    rows[b], g_hbm.at[pl.ds(cs[b] * CHUNK, CHUNK)], sem_w).wait()
        return carry

    lax.fori_loop(0, GSTEPS, grp, 0)


def _sc_scatter_body(z_hbm, pf_hbm, out_hbm, idx_v, idx_v2, rows_v, rows_v2,
                     acc_sh, sem_z, sem_s):
    cid = lax.axis_index("c")
    sid = lax.axis_index("s")
    wid = cid * 16 + sid

    zeros16 = jnp.zeros((16,), jnp.float32)

    def zstep(i, carry):
        r = i // (NF // 16)
        col = (i % (NF // 16)) * 16
        rows_v[r, pl.ds(col, 16)] = zeros16
        return carry

    lax.fori_loop(0, CHUNK * (NF // 16), zstep, 0)

    for i in range((NFULL + 15) // 16):
        blk = sid + i * 16

        @pl.when(blk < NFULL)
        def _():
            pltpu.sync_copy(rows_v, acc_sh.at[pl.ds(blk * CHUNK, CHUNK)])

    @pl.when(sid == 0)
    def _():
        pltpu.sync_copy(rows_v.at[pl.ds(0, NREM)],
                        acc_sh.at[pl.ds(NFULL * CHUNK, NREM)])

    plsc.subcore_barrier()

    idxs = [idx_v, idx_v2]
    rows = [rows_v, rows_v2]

    def grp(jj, carry):
        cs = [(jj * 2 + b) * NW + wid for b in range(2)]
        for b in range(2):
            @pl.when(cs[b] < NCHUNKS)
            def _(b=b):
                pltpu.sync_copy(pf_hbm.at[pl.ds(cs[b] * CHUNK, CHUNK)], idxs[b])
                pltpu.async_copy(
                    z_hbm.at[pl.ds(cs[b] * CHUNK, CHUNK)], rows[b], sem_z)
        for b in range(2):
            @pl.when(cs[b] < NCHUNKS)
            def _(b=b):
                pltpu.make_async_copy(
                    z_hbm.at[pl.ds(cs[b] * CHUNK, CHUNK)], rows[b], sem_z).wait()
        for b in range(2):
            @pl.when(cs[b] < NCHUNKS)
            def _(b=b):
                pltpu.async_copy(rows[b], acc_sh.at[idxs[b]], sem_s, add=True)
        for b in range(2):
            @pl.when(cs[b] < NCHUNKS)
            def _(b=b):
                pltpu.make_async_copy(
                    rows[b], acc_sh.at[idxs[b]], sem_s).wait()
        return carry

    lax.fori_loop(0, (STEPS + 1) // 2, grp, 0)
    plsc.subcore_barrier()

    for i in range((NFULL + 15) // 16):
        blk = sid + i * 16

        @pl.when(blk < NFULL)
        def _():
            pltpu.sync_copy(acc_sh.at[pl.ds(blk * CHUNK, CHUNK)], rows_v)
            pltpu.sync_copy(rows_v, out_hbm.at[pl.ds(cid * N + blk * CHUNK, CHUNK)])

    @pl.when(sid == 0)
    def _():
        pltpu.sync_copy(acc_sh.at[pl.ds(NFULL * CHUNK, NREM)],
                        rows_v.at[pl.ds(0, NREM)])
        pltpu.sync_copy(rows_v.at[pl.ds(0, NREM)],
                        out_hbm.at[pl.ds(cid * N + NFULL * CHUNK, NREM)])


def _tc_z_body(g_ref, d_ref, w_ref, mu_ref, sg_ref, z_ref):
    gt = jnp.transpose(g_ref[...])
    d = d_ref[0]
    inv = 1.0 / d
    cut = jnp.where(
        d < HARD_CUTOFF,
        jnp.cos(d * (jnp.pi / (2.0 * HARD_CUTOFF))) ** 2,
        0.0,
    )
    rows = []
    for k in range(ND):
        t = (inv - mu_ref[0, k]) / sg_ref[0, k]
        rows.append(jnp.exp(-0.5 * t * t) * cut)
    kr = jnp.concatenate([rows[k] * gt for k in range(ND)], axis=0)
    zt = jnp.dot(w_ref[...], kr, preferred_element_type=jnp.float32)
    z_ref[...] = jnp.transpose(zt)


TD = 400


def _tc_out_body(p1_ref, p2_ref, p3_ref, p4_ref, p5_ref, x_ref, w_ref, b_ref,
                 o_ref):
    s = jnp.dot(x_ref[...], w_ref[...], preferred_element_type=jnp.float32)
    o_ref[...] = (((p1_ref[0] + p1_ref[1]) + (p2_ref[0] + p2_ref[1]))
                  + ((p3_ref[0] + p3_ref[1]) + (p4_ref[0] + p4_ref[1]))
                  + (p5_ref[0] + p5_ref[1]) + s + b_ref[...])


def kernel(in_features, pair_first, pair_second, dist_pairs, mu, sigma,
           int_weights, self_W, self_b):
    ps = pair_second.astype(jnp.int32)
    pf = pair_first.astype(jnp.int32)
    x = in_features.astype(jnp.float32)

    mesh = plsc.VectorSubcoreMesh(core_axis_name="c", subcore_axis_name="s")

    gather = pl.kernel(
        _sc_gather_body,
        out_type=jax.ShapeDtypeStruct((EP, NF), jnp.float32),
        mesh=mesh,
        scratch_types=(
            [pltpu.VMEM((CHUNK,), jnp.int32)] * GG
            + [pltpu.VMEM((CHUNK, NF), jnp.float32)] * GG
            + [pltpu.SemaphoreType.DMA] * 3
        ),
    )

    wm = jnp.transpose(int_weights, (1, 0, 2)).reshape(NF, ND * NF)
    mu2 = mu.astype(jnp.float32).reshape(1, ND)
    sg2 = sigma.astype(jnp.float32).reshape(1, ND)

    def tc_z(g_part, dist_part):
        return pl.pallas_call(
            _tc_z_body,
            grid=(NB,),
            in_specs=[
                pl.BlockSpec((BB, NF), lambda b: (b, 0)),
                pl.BlockSpec((1, 1, BB), lambda b: (b, 0, 0)),
                pl.BlockSpec((NF, ND * NF), lambda b: (0, 0)),
                pl.BlockSpec(memory_space=pltpu.SMEM),
                pl.BlockSpec(memory_space=pltpu.SMEM),
            ],
            out_specs=pl.BlockSpec((BB, NF), lambda b: (b, 0)),
            out_shape=jax.ShapeDtypeStruct((EP, NF), jnp.float32),
        )(g_part, dist_part.reshape(NB, 1, BB), wm, mu2, sg2)

    scatter = pl.kernel(
        _sc_scatter_body,
        out_type=jax.ShapeDtypeStruct((2 * N, NF), jnp.float32),
        mesh=mesh,
        scratch_types=[
            pltpu.VMEM((CHUNK,), jnp.int32),
            pltpu.VMEM((CHUNK,), jnp.int32),
            pltpu.VMEM((CHUNK, NF), jnp.float32),
            pltpu.VMEM((CHUNK, NF), jnp.float32),
            pltpu.VMEM_SHARED((N, NF), jnp.float32),
            pltpu.SemaphoreType.DMA,
            pltpu.SemaphoreType.DMA,
        ],
    )

    dist = dist_pairs.astype(jnp.float32)
    gs = [gather(x, ps[i * EP:(i + 1) * EP]) for i in range(PARTS)]
    zs = [tc_z(gs[i], dist[i * EP:(i + 1) * EP]) for i in range(PARTS)]
    pps = [scatter(zs[i], pf[i * EP:(i + 1) * EP]).reshape(2, N, NF)
           for i in range(PARTS)]

    swt = jnp.transpose(self_W, (1, 0)).astype(jnp.float32)
    b2 = self_b.astype(jnp.float32).reshape(1, NF)
    out = pl.pallas_call(
        _tc_out_body,
        grid=(N // TD,),
        in_specs=[
            pl.BlockSpec((2, TD, NF), lambda b: (0, b, 0)),
            pl.BlockSpec((2, TD, NF), lambda b: (0, b, 0)),
            pl.BlockSpec((2, TD, NF), lambda b: (0, b, 0)),
            pl.BlockSpec((2, TD, NF), lambda b: (0, b, 0)),
            pl.BlockSpec((2, TD, NF), lambda b: (0, b, 0)),
            pl.BlockSpec((TD, NF), lambda b: (b, 0)),
            pl.BlockSpec((NF, NF), lambda b: (0, 0)),
            pl.BlockSpec((1, NF), lambda b: (0, 0)),
        ],
        out_specs=pl.BlockSpec((TD, NF), lambda b: (b, 0)),
        out_shape=jax.ShapeDtypeStruct((N, NF), jnp.float32),
    )(pps[0], pps[1], pps[2], pps[3], pps[4], x, swt, b2)
    return out
